# Initial kernel scaffold; baseline (speedup 1.0000x reference)
import functools

import jax
import jax.numpy as jnp
from jax import lax
from jax.experimental import pallas as pl
from jax.experimental.pallas import tpu as pltpu
from jax.experimental.pallas import tpu_sc as plsc

_NC = 2
_NS = 16
_NW = _NC * _NS

_BE = 4000
_BN = 2000
_C = 80


def _sc_gather_pair(table, src, dst):
  n, d = table.shape
  e = src.shape[0]
  epw = e // _NW
  steps = epw // _C
  mesh = plsc.VectorSubcoreMesh(core_axis_name="c", subcore_axis_name="s")

  @functools.partial(
      pl.kernel,
      out_type=(jax.ShapeDtypeStruct((e, d), jnp.float32),
                jax.ShapeDtypeStruct((e, d), jnp.float32)),
      mesh=mesh,
      scratch_types=[
          pltpu.VMEM((_C,), jnp.int32),
          pltpu.VMEM((_C,), jnp.int32),
          pltpu.VMEM((_C, d), jnp.float32),
          pltpu.VMEM((_C, d), jnp.float32),
          pltpu.SemaphoreType.DMA,
          pltpu.SemaphoreType.DMA,
      ],
  )
  def k(table_h, src_h, dst_h, oa_h, ob_h, ia_v, ib_v, ra_v, rb_v, sa, sb):
    wid = lax.axis_index("s") * _NC + lax.axis_index("c")
    base = wid * epw

    def step(i, carry):
      off = base + i * _C
      pltpu.sync_copy(src_h.at[pl.ds(off, _C)], ia_v)
      pltpu.sync_copy(dst_h.at[pl.ds(off, _C)], ib_v)
      cpa = pltpu.async_copy(table_h.at[ia_v], ra_v, sa)
      cpb = pltpu.async_copy(table_h.at[ib_v], rb_v, sb)
      cpa.wait()
      cpb.wait()
      pltpu.sync_copy(ra_v, oa_h.at[pl.ds(off, _C)])
      pltpu.sync_copy(rb_v, ob_h.at[pl.ds(off, _C)])
      return carry

    lax.fori_loop(0, steps, step, 0)

  return k(table, src, dst)


def _sc_segsum(feat, dst, n):
  e, d = feat.shape
  epw = e // _NW
  steps = epw // _C
  rps = n // _NS
  zrows = 125
  mesh = plsc.VectorSubcoreMesh(core_axis_name="c", subcore_axis_name="s")

  @functools.partial(
      pl.kernel,
      out_type=(jax.ShapeDtypeStruct((2 * n, d), jnp.float32),
                jax.ShapeDtypeStruct((2 * n, 16), jnp.float32)),
      mesh=mesh,
      scratch_types=[
          pltpu.VMEM((_C,), jnp.int32),
          pltpu.VMEM((_C, d), jnp.float32),
          pltpu.VMEM((_C, 16), jnp.float32),
          pltpu.VMEM((125, d), jnp.float32),
          pltpu.VMEM((n // _NS, 16), jnp.float32),
          pltpu.VMEM_SHARED((n, d), jnp.float32),
          pltpu.VMEM_SHARED((n, 16), jnp.float32),
      ],
  )
  def k(feat_h, dst_h, sums_h, cnts_h,
        idx_v, rows_v, ones_v, zbuf_v, czbuf_v, acc_s, cnt_s):
    cid = lax.axis_index("c")
    sid = lax.axis_index("s")
    wid = sid * _NC + cid

    zero16 = jnp.zeros((16,), jnp.float32)
    one16 = jnp.ones((16,), jnp.float32)

    def fill_z(r, carry):
      for kk in range(d // 16):
        zbuf_v[r, pl.ds(16 * kk, 16)] = zero16
      return carry

    lax.fori_loop(0, zrows, fill_z, 0)

    def fill_cz(r, carry):
      czbuf_v[r, pl.ds(0, 16)] = zero16
      return carry

    lax.fori_loop(0, rps, fill_cz, 0)

    def fill_o(r, carry):
      ones_v[r, pl.ds(0, 16)] = one16
      return carry

    lax.fori_loop(0, _C, fill_o, 0)

    rbase = sid * rps
    for j in range(rps // zrows):
      pltpu.sync_copy(zbuf_v, acc_s.at[pl.ds(rbase + j * zrows, zrows)])
    pltpu.sync_copy(czbuf_v, cnt_s.at[pl.ds(rbase, rps)])

    plsc.subcore_barrier()

    base = wid * epw

    def step(i, carry):
      off = base + i * _C
      pltpu.sync_copy(dst_h.at[pl.ds(off, _C)], idx_v)
      pltpu.sync_copy(feat_h.at[pl.ds(off, _C)], rows_v)
      pltpu.sync_copy(rows_v, acc_s.at[idx_v], add=True)
      pltpu.sync_copy(ones_v, cnt_s.at[idx_v], add=True)
      return carry

    lax.fori_loop(0, steps, step, 0)

    plsc.subcore_barrier()

    obase = cid * n + rbase
    pltpu.sync_copy(acc_s.at[pl.ds(rbase, rps)], sums_h.at[pl.ds(obase, rps)])
    pltpu.sync_copy(cnt_s.at[pl.ds(rbase, rps)], cnts_h.at[pl.ds(obase, rps)])

  return k(feat, dst)


def _dot(a, b):
  return jnp.dot(a, b, preferred_element_type=jnp.float32)


def _tc_embed(h, we, be, w1n, b1n, a_s):
  n, f = h.shape
  d = we.shape[1]
  grid = n // _BN

  def body(a_ref, h_ref, we_ref, be_ref, w1_ref, b1_ref,
           h0_ref, fn1_ref, fn1s_ref):
    h0 = _dot(h_ref[...], we_ref[...]) + be_ref[...]
    h0_ref[...] = h0
    fn1 = jnp.maximum(_dot(h0, w1_ref[...]) + b1_ref[...], 0.0)
    fn1_ref[...] = fn1
    fn1s_ref[...] = fn1 * a_ref[0, 0]

  return pl.pallas_call(
      body,
      grid=(grid,),
      in_specs=[
          pl.BlockSpec(memory_space=pltpu.SMEM),
          pl.BlockSpec((_BN, f), lambda i: (i, 0)),
          pl.BlockSpec((f, d), lambda i: (0, 0)),
          pl.BlockSpec((1, d), lambda i: (0, 0)),
          pl.BlockSpec((d, d), lambda i: (0, 0)),
          pl.BlockSpec((1, d), lambda i: (0, 0)),
      ],
      out_specs=[pl.BlockSpec((_BN, d), lambda i: (i, 0))] * 3,
      out_shape=[jax.ShapeDtypeStruct((n, d), jnp.float32)] * 3,
  )(a_s, h, we, be.reshape(1, d), w1n, b1n.reshape(1, d))


def _tc_mm1(ga, gb, w, b):
  e, d = ga.shape
  grid = e // _BE

  def body(ga_ref, gb_ref, w_ref, b_ref, o_ref):
    x = ga_ref[...] + gb_ref[...]
    o_ref[...] = jnp.maximum(_dot(x, w_ref[...]) + b_ref[...], 0.0)

  return pl.pallas_call(
      body,
      grid=(grid,),
      in_specs=[
          pl.BlockSpec((_BE, d), lambda i: (i, 0)),
          pl.BlockSpec((_BE, d), lambda i: (i, 0)),
          pl.BlockSpec((d, d), lambda i: (0, 0)),
          pl.BlockSpec((1, d), lambda i: (0, 0)),
      ],
      out_specs=pl.BlockSpec((_BE, d), lambda i: (i, 0)),
      out_shape=jax.ShapeDtypeStruct((e, d), jnp.float32),
  )(ga, gb, w, b.reshape(1, d))


def _tc_mm2(s, ga, gb, fe, w, b):
  e, d = ga.shape
  grid = e // _BE

  def body(s_ref, ga_ref, gb_ref, fe_ref, w_ref, b_ref, o_ref):
    x = ga_ref[...] + gb_ref[...] + s_ref[0, 0] * fe_ref[...]
    o_ref[...] = jnp.maximum(_dot(x, w_ref[...]) + b_ref[...], 0.0)

  return pl.pallas_call(
      body,
      grid=(grid,),
      in_specs=[
          pl.BlockSpec(memory_space=pltpu.SMEM),
          pl.BlockSpec((_BE, d), lambda i: (i, 0)),
          pl.BlockSpec((_BE, d), lambda i: (i, 0)),
          pl.BlockSpec((_BE, d), lambda i: (i, 0)),
          pl.BlockSpec((d, d), lambda i: (0, 0)),
          pl.BlockSpec((1, d), lambda i: (0, 0)),
      ],
      out_specs=pl.BlockSpec((_BE, d), lambda i: (i, 0)),
      out_shape=jax.ShapeDtypeStruct((e, d), jnp.float32),
  )(s, ga, gb, fe, w, b.reshape(1, d))


def _mlp56(x, w1, b1, w2, b2, w3, b3, w4, b4, wo, bo):
  x = jnp.maximum(_dot(x, w1) + b1, 0.0)
  x = jnp.maximum(_dot(x, w2) + b2, 0.0)
  x = jnp.maximum(_dot(x, w3) + b3, 0.0)
  x = jnp.maximum(_dot(x, w4) + b4, 0.0)
  return _dot(x, wo) + bo


def _tc_mlp56_edges(s, ga, gb, fe, w1, b1, w2, b2, w3, b3, w4, b4, wo, bo):
  e, d = ga.shape
  nc = wo.shape[1]
  grid = e // _BE

  def body(s_ref, ga_ref, gb_ref, fe_ref,
           w1_ref, b1_ref, w2_ref, b2_ref, w3_ref, b3_ref, w4_ref, b4_ref,
           wo_ref, bo_ref, o_ref):
    x = ga_ref[...] + gb_ref[...] + s_ref[0, 0] * fe_ref[...]
    o_ref[...] = _mlp56(x, w1_ref[...], b1_ref[...], w2_ref[...], b2_ref[...],
                        w3_ref[...], b3_ref[...], w4_ref[...], b4_ref[...],
                        wo_ref[...], bo_ref[...])

  wspec = pl.BlockSpec((d, d), lambda i: (0, 0))
  bspec = pl.BlockSpec((1, d), lambda i: (0, 0))
  espec = pl.BlockSpec((_BE, d), lambda i: (i, 0))
  return pl.pallas_call(
      body,
      grid=(grid,),
      in_specs=[
          pl.BlockSpec(memory_space=pltpu.SMEM),
          espec, espec, espec,
          wspec, bspec, wspec, bspec, wspec, bspec, wspec, bspec,
          pl.BlockSpec((d, nc), lambda i: (0, 0)),
          pl.BlockSpec((1, nc), lambda i: (0, 0)),
      ],
      out_specs=pl.BlockSpec((_BE, nc), lambda i: (i, 0)),
      out_shape=jax.ShapeDtypeStruct((e, nc), jnp.float32),
  )(s, ga, gb, fe, w1, b1.reshape(1, d), w2, b2.reshape(1, d),
    w3, b3.reshape(1, d), w4, b4.reshape(1, d), wo, bo.reshape(1, nc))


def _tc_node_mid(scal, fn1, sums, cnts, w3n, b3n):
  n, d = fn1.shape
  grid = n // _BN
  nblocks = n // _BN

  def body(s_ref, fn1_ref, s0_ref, s1_ref, c0_ref, c1_ref, w_ref, b_ref,
           fn3_ref, fn3s_ref):
    cnt = c0_ref[:, 0:1] + c1_ref[:, 0:1]
    segm = (s0_ref[...] + s1_ref[...]) / jnp.maximum(cnt, 1.0)
    new_n = s_ref[0, 0] * fn1_ref[...] + s_ref[0, 1] * segm
    fn3 = jnp.maximum(_dot(new_n, w_ref[...]) + b_ref[...], 0.0)
    fn3_ref[...] = fn3
    fn3s_ref[...] = fn3 * s_ref[0, 2]

  return pl.pallas_call(
      body,
      grid=(grid,),
      in_specs=[
          pl.BlockSpec(memory_space=pltpu.SMEM),
          pl.BlockSpec((_BN, d), lambda i: (i, 0)),
          pl.BlockSpec((_BN, d), lambda i: (i, 0)),
          pl.BlockSpec((_BN, d), lambda i, nb=nblocks: (nb + i, 0)),
          pl.BlockSpec((_BN, 16), lambda i: (i, 0)),
          pl.BlockSpec((_BN, 16), lambda i, nb=nblocks: (nb + i, 0)),
          pl.BlockSpec((d, d), lambda i: (0, 0)),
          pl.BlockSpec((1, d), lambda i: (0, 0)),
      ],
      out_specs=[pl.BlockSpec((_BN, d), lambda i: (i, 0))] * 2,
      out_shape=[jax.ShapeDtypeStruct((n, d), jnp.float32)] * 2,
  )(scal, fn1, sums, sums, cnts, cnts, w3n, b3n.reshape(1, d))


def _tc_node_final(scal, fn3, sums, cnts,
                   w1, b1, w2, b2, w3, b3, w4, b4, wo, bo):
  n, d = fn3.shape
  nc = wo.shape[1]
  grid = n // _BN
  nblocks = n // _BN

  def body(s_ref, fn3_ref, s0_ref, s1_ref, c0_ref, c1_ref,
           w1_ref, b1_ref, w2_ref, b2_ref, w3_ref, b3_ref, w4_ref, b4_ref,
           wo_ref, bo_ref, o_ref):
    cnt = c0_ref[:, 0:1] + c1_ref[:, 0:1]
    segm = (s0_ref[...] + s1_ref[...]) / jnp.maximum(cnt, 1.0)
    new_n = s_ref[0, 0] * fn3_ref[...] + s_ref[0, 1] * segm
    o_ref[...] = _mlp56(new_n, w1_ref[...], b1_ref[...], w2_ref[...],
                        b2_ref[...], w3_ref[...], b3_ref[...], w4_ref[...],
                        b4_ref[...], wo_ref[...], bo_ref[...])

  wspec = pl.BlockSpec((d, d), lambda i: (0, 0))
  bspec = pl.BlockSpec((1, d), lambda i: (0, 0))
  return pl.pallas_call(
      body,
      grid=(grid,),
      in_specs=[
          pl.BlockSpec(memory_space=pltpu.SMEM),
          pl.BlockSpec((_BN, d), lambda i: (i, 0)),
          pl.BlockSpec((_BN, d), lambda i: (i, 0)),
          pl.BlockSpec((_BN, d), lambda i, nb=nblocks: (nb + i, 0)),
          pl.BlockSpec((_BN, 16), lambda i: (i, 0)),
          pl.BlockSpec((_BN, 16), lambda i, nb=nblocks: (nb + i, 0)),
          wspec, bspec, wspec, bspec, wspec, bspec, wspec, bspec,
          pl.BlockSpec((d, nc), lambda i: (0, 0)),
          pl.BlockSpec((1, nc), lambda i: (0, 0)),
      ],
      out_specs=pl.BlockSpec((_BN, nc), lambda i: (i, 0)),
      out_shape=jax.ShapeDtypeStruct((n, nc), jnp.float32),
  )(scal, fn3, sums, sums, cnts, cnts, w1, b1.reshape(1, d), w2,
    b2.reshape(1, d), w3, b3.reshape(1, d), w4, b4.reshape(1, d),
    wo, bo.reshape(1, nc))


def kernel(h, edge_index, W_embed, b_embed, W1n, b1n, W1e, b1e,
           w2_nn, w2_ne, w2_en, w2_ee, W3n, b3n, W3e, b3e,
           w4_nn, w4_ne, w4_en, w4_ee, W5_1, b5_1, W5_2, b5_2,
           W5_3, b5_3, W5_4, b5_4, W5_out, b5_out):
  n = h.shape[0]
  src = edge_index[0]
  dst = edge_index[1]

  a2 = (0.5 * w2_en).reshape(1, 1)
  h0, fn1, fn1s = _tc_embed(h, W_embed, b_embed, W1n, b1n, a2)

  ga1, gb1 = _sc_gather_pair(h0, src, dst)
  fe1 = _tc_mm1(ga1, gb1, 0.5 * W1e, b1e)
  sums1, cnts = _sc_segsum(fe1, dst, n)

  scal_mid = jnp.concatenate([w2_nn, w2_ne, 0.5 * w4_en]).reshape(1, 3)
  fn3, fn3s = _tc_node_mid(scal_mid, fn1, sums1, cnts, W3n, b3n)

  ga2, gb2 = _sc_gather_pair(fn1s, src, dst)
  fe3 = _tc_mm2(w2_ee.reshape(1, 1), ga2, gb2, fe1, W3e, b3e)
  sums2, _ = _sc_segsum(fe3, dst, n)

  ga3, gb3 = _sc_gather_pair(fn3s, src, dst)
  out_e = _tc_mlp56_edges(w4_ee.reshape(1, 1), ga3, gb3, fe3,
                          W5_1, b5_1, W5_2, b5_2, W5_3, b5_3, W5_4, b5_4,
                          W5_out, b5_out)

  scal_fin = jnp.concatenate([w4_nn, w4_ne]).reshape(1, 2)
  out_n = _tc_node_final(scal_fin, fn3, sums2, cnts,
                         W5_1, b5_1, W5_2, b5_2, W5_3, b5_3, W5_4, b5_4,
                         W5_out, b5_out)
  return out_n, out_e


# speedup vs baseline: 2.4514x; 2.4514x over previous
import functools

import jax
import jax.numpy as jnp
from jax import lax
from jax.experimental import pallas as pl
from jax.experimental.pallas import tpu as pltpu
from jax.experimental.pallas import tpu_sc as plsc

_NC = 2
_NS = 16
_NW = _NC * _NS

_BE = 4000
_BN = 2000
_C = 80


def _sc_gather_pair(table, src, dst):
  n, d = table.shape
  e = src.shape[0]
  epw = e // _NW
  steps = epw // _C
  mesh = plsc.VectorSubcoreMesh(core_axis_name="c", subcore_axis_name="s")

  @functools.partial(
      pl.kernel,
      out_type=(jax.ShapeDtypeStruct((e, d), jnp.float32),
                jax.ShapeDtypeStruct((e, d), jnp.float32)),
      mesh=mesh,
      scratch_types=[
          pltpu.VMEM((_C,), jnp.int32),
          pltpu.VMEM((_C,), jnp.int32),
          pltpu.VMEM((_C, d), jnp.float32),
          pltpu.VMEM((_C, d), jnp.float32),
          pltpu.SemaphoreType.DMA,
          pltpu.SemaphoreType.DMA,
      ],
  )
  def k(table_h, src_h, dst_h, oa_h, ob_h, ia_v, ib_v, ra_v, rb_v, sa, sb):
    wid = lax.axis_index("s") * _NC + lax.axis_index("c")
    base = wid * epw

    def step(i, carry):
      off = base + i * _C
      pltpu.sync_copy(src_h.at[pl.ds(off, _C)], ia_v)
      pltpu.sync_copy(dst_h.at[pl.ds(off, _C)], ib_v)
      cpa = pltpu.async_copy(table_h.at[ia_v], ra_v, sa)
      cpb = pltpu.async_copy(table_h.at[ib_v], rb_v, sb)
      cpa.wait()
      cpb.wait()
      pltpu.sync_copy(ra_v, oa_h.at[pl.ds(off, _C)])
      pltpu.sync_copy(rb_v, ob_h.at[pl.ds(off, _C)])
      return carry

    lax.fori_loop(0, steps, step, 0)

  return k(table, src, dst)


def _sc_segsum(feat, dst, n):
  e, d = feat.shape
  epw = e // _NW
  steps = epw // _C
  rps = (n // (_NS * 8)) * 8
  tail = n - rps * _NS
  zrows = rps // 3
  mesh = plsc.VectorSubcoreMesh(core_axis_name="c", subcore_axis_name="s")

  @functools.partial(
      pl.kernel,
      out_type=jax.ShapeDtypeStruct((2 * n, d), jnp.float32),
      mesh=mesh,
      scratch_types=[
          pltpu.VMEM((_C,), jnp.int32),
          pltpu.VMEM((_C, d), jnp.float32),
          pltpu.VMEM((zrows, d), jnp.float32),
          pltpu.VMEM_SHARED((n, d), jnp.float32),
      ],
  )
  def k(feat_h, dst_h, sums_h, idx_v, rows_v, zbuf_v, acc_s):
    cid = lax.axis_index("c")
    sid = lax.axis_index("s")
    wid = sid * _NC + cid

    zero16 = jnp.zeros((16,), jnp.float32)

    def fill_z(r, carry):
      for kk in range(d // 16):
        zbuf_v[r, pl.ds(16 * kk, 16)] = zero16
      return carry

    lax.fori_loop(0, zrows, fill_z, 0)

    rbase = sid * rps
    for j in range(rps // zrows):
      pltpu.sync_copy(zbuf_v, acc_s.at[pl.ds(rbase + j * zrows, zrows)])

    @pl.when(sid == _NS - 1)
    def _zero_tail():
      pltpu.sync_copy(zbuf_v.at[pl.ds(0, tail)],
                      acc_s.at[pl.ds(rps * _NS, tail)])

    plsc.subcore_barrier()

    base = wid * epw

    def step(i, carry):
      off = base + i * _C
      pltpu.sync_copy(dst_h.at[pl.ds(off, _C)], idx_v)
      pltpu.sync_copy(feat_h.at[pl.ds(off, _C)], rows_v)
      pltpu.sync_copy(rows_v, acc_s.at[idx_v], add=True)
      return carry

    lax.fori_loop(0, steps, step, 0)

    plsc.subcore_barrier()

    obase = cid * n + rbase
    pltpu.sync_copy(acc_s.at[pl.ds(rbase, rps)], sums_h.at[pl.ds(obase, rps)])

    @pl.when(sid == _NS - 1)
    def _write_tail():
      tbase = rps * _NS
      pltpu.sync_copy(acc_s.at[pl.ds(tbase, tail)],
                      sums_h.at[pl.ds(cid * n + tbase, tail)])

  return k(feat, dst)


def _sc_counts(dst, n):
  e = dst.shape[0]
  epw = e // _NW
  steps = epw // _C
  d = 128
  rps = (n // (_NS * 8)) * 8
  tail = n - rps * _NS
  zrows = rps // 3
  mesh = plsc.VectorSubcoreMesh(core_axis_name="c", subcore_axis_name="s")

  @functools.partial(
      pl.kernel,
      out_type=jax.ShapeDtypeStruct((2 * n, d), jnp.float32),
      mesh=mesh,
      scratch_types=[
          pltpu.VMEM((_C,), jnp.int32),
          pltpu.VMEM((_C, d), jnp.float32),
          pltpu.VMEM((zrows, d), jnp.float32),
          pltpu.VMEM_SHARED((n, d), jnp.float32),
      ],
  )
  def k(dst_h, cnts_h, idx_v, ones_v, zbuf_v, cnt_s):
    cid = lax.axis_index("c")
    sid = lax.axis_index("s")
    wid = sid * _NC + cid

    zero16 = jnp.zeros((16,), jnp.float32)
    one16 = jnp.ones((16,), jnp.float32)

    def fill_z(r, carry):
      for kk in range(d // 16):
        zbuf_v[r, pl.ds(16 * kk, 16)] = zero16
      return carry

    lax.fori_loop(0, zrows, fill_z, 0)

    def fill_o(r, carry):
      for kk in range(d // 16):
        ones_v[r, pl.ds(16 * kk, 16)] = one16
      return carry

    lax.fori_loop(0, _C, fill_o, 0)

    rbase = sid * rps
    for j in range(rps // zrows):
      pltpu.sync_copy(zbuf_v, cnt_s.at[pl.ds(rbase + j * zrows, zrows)])

    @pl.when(sid == _NS - 1)
    def _zero_tail():
      pltpu.sync_copy(zbuf_v.at[pl.ds(0, tail)],
                      cnt_s.at[pl.ds(rps * _NS, tail)])

    plsc.subcore_barrier()

    base = wid * epw

    def step(i, carry):
      off = base + i * _C
      pltpu.sync_copy(dst_h.at[pl.ds(off, _C)], idx_v)
      pltpu.sync_copy(ones_v, cnt_s.at[idx_v], add=True)
      return carry

    lax.fori_loop(0, steps, step, 0)

    plsc.subcore_barrier()

    obase = cid * n + rbase
    pltpu.sync_copy(cnt_s.at[pl.ds(rbase, rps)], cnts_h.at[pl.ds(obase, rps)])

    @pl.when(sid == _NS - 1)
    def _write_tail():
      tbase = rps * _NS
      pltpu.sync_copy(cnt_s.at[pl.ds(tbase, tail)],
                      cnts_h.at[pl.ds(cid * n + tbase, tail)])

  return k(dst)


def _dot(a, b):
  return jnp.dot(a, b, preferred_element_type=jnp.float32)


def _tc_embed(h, we, be, w1n, b1n, a_s):
  n, f = h.shape
  d = we.shape[1]
  grid = n // _BN

  def body(a_ref, h_ref, we_ref, be_ref, w1_ref, b1_ref,
           h0_ref, fn1_ref, fn1s_ref):
    h0 = _dot(h_ref[...], we_ref[...]) + be_ref[...]
    h0_ref[...] = h0
    fn1 = jnp.maximum(_dot(h0, w1_ref[...]) + b1_ref[...], 0.0)
    fn1_ref[...] = fn1
    fn1s_ref[...] = fn1 * a_ref[0, 0]

  return pl.pallas_call(
      body,
      grid=(grid,),
      in_specs=[
          pl.BlockSpec(memory_space=pltpu.SMEM),
          pl.BlockSpec((_BN, f), lambda i: (i, 0)),
          pl.BlockSpec((f, d), lambda i: (0, 0)),
          pl.BlockSpec((1, d), lambda i: (0, 0)),
          pl.BlockSpec((d, d), lambda i: (0, 0)),
          pl.BlockSpec((1, d), lambda i: (0, 0)),
      ],
      out_specs=[pl.BlockSpec((_BN, d), lambda i: (i, 0))] * 3,
      out_shape=[jax.ShapeDtypeStruct((n, d), jnp.float32)] * 3,
  )(a_s, h, we, be.reshape(1, d), w1n, b1n.reshape(1, d))


def _tc_mm1(ga, gb, w, b):
  e, d = ga.shape
  grid = e // _BE

  def body(ga_ref, gb_ref, w_ref, b_ref, o_ref):
    x = ga_ref[...] + gb_ref[...]
    o_ref[...] = jnp.maximum(_dot(x, w_ref[...]) + b_ref[...], 0.0)

  return pl.pallas_call(
      body,
      grid=(grid,),
      in_specs=[
          pl.BlockSpec((_BE, d), lambda i: (i, 0)),
          pl.BlockSpec((_BE, d), lambda i: (i, 0)),
          pl.BlockSpec((d, d), lambda i: (0, 0)),
          pl.BlockSpec((1, d), lambda i: (0, 0)),
      ],
      out_specs=pl.BlockSpec((_BE, d), lambda i: (i, 0)),
      out_shape=jax.ShapeDtypeStruct((e, d), jnp.float32),
  )(ga, gb, w, b.reshape(1, d))


def _tc_mm2(s, ga, gb, fe, w, b):
  e, d = ga.shape
  grid = e // _BE

  def body(s_ref, ga_ref, gb_ref, fe_ref, w_ref, b_ref, o_ref):
    x = ga_ref[...] + gb_ref[...] + s_ref[0, 0] * fe_ref[...]
    o_ref[...] = jnp.maximum(_dot(x, w_ref[...]) + b_ref[...], 0.0)

  return pl.pallas_call(
      body,
      grid=(grid,),
      in_specs=[
          pl.BlockSpec(memory_space=pltpu.SMEM),
          pl.BlockSpec((_BE, d), lambda i: (i, 0)),
          pl.BlockSpec((_BE, d), lambda i: (i, 0)),
          pl.BlockSpec((_BE, d), lambda i: (i, 0)),
          pl.BlockSpec((d, d), lambda i: (0, 0)),
          pl.BlockSpec((1, d), lambda i: (0, 0)),
      ],
      out_specs=pl.BlockSpec((_BE, d), lambda i: (i, 0)),
      out_shape=jax.ShapeDtypeStruct((e, d), jnp.float32),
  )(s, ga, gb, fe, w, b.reshape(1, d))


def _mlp56(x, w1, b1, w2, b2, w3, b3, w4, b4, wo, bo):
  x = jnp.maximum(_dot(x, w1) + b1, 0.0)
  x = jnp.maximum(_dot(x, w2) + b2, 0.0)
  x = jnp.maximum(_dot(x, w3) + b3, 0.0)
  x = jnp.maximum(_dot(x, w4) + b4, 0.0)
  return _dot(x, wo) + bo


def _tc_mlp56_edges(s, ga, gb, fe, w1, b1, w2, b2, w3, b3, w4, b4, wo, bo):
  e, d = ga.shape
  nc = wo.shape[1]
  grid = e // _BE

  def body(s_ref, ga_ref, gb_ref, fe_ref,
           w1_ref, b1_ref, w2_ref, b2_ref, w3_ref, b3_ref, w4_ref, b4_ref,
           wo_ref, bo_ref, o_ref):
    x = ga_ref[...] + gb_ref[...] + s_ref[0, 0] * fe_ref[...]
    o_ref[...] = _mlp56(x, w1_ref[...], b1_ref[...], w2_ref[...], b2_ref[...],
                        w3_ref[...], b3_ref[...], w4_ref[...], b4_ref[...],
                        wo_ref[...], bo_ref[...])

  wspec = pl.BlockSpec((d, d), lambda i: (0, 0))
  bspec = pl.BlockSpec((1, d), lambda i: (0, 0))
  espec = pl.BlockSpec((_BE, d), lambda i: (i, 0))
  return pl.pallas_call(
      body,
      grid=(grid,),
      in_specs=[
          pl.BlockSpec(memory_space=pltpu.SMEM),
          espec, espec, espec,
          wspec, bspec, wspec, bspec, wspec, bspec, wspec, bspec,
          pl.BlockSpec((d, nc), lambda i: (0, 0)),
          pl.BlockSpec((1, nc), lambda i: (0, 0)),
      ],
      out_specs=pl.BlockSpec((_BE, nc), lambda i: (i, 0)),
      out_shape=jax.ShapeDtypeStruct((e, nc), jnp.float32),
  )(s, ga, gb, fe, w1, b1.reshape(1, d), w2, b2.reshape(1, d),
    w3, b3.reshape(1, d), w4, b4.reshape(1, d), wo, bo.reshape(1, nc))


def _tc_node_mid(scal, fn1, sums, cnts, w3n, b3n):
  n, d = fn1.shape
  grid = n // _BN
  nblocks = n // _BN

  def body(s_ref, fn1_ref, s0_ref, s1_ref, c0_ref, c1_ref, w_ref, b_ref,
           fn3_ref, fn3s_ref):
    cnt = c0_ref[:, 0:1] + c1_ref[:, 0:1]
    segm = (s0_ref[...] + s1_ref[...]) / jnp.maximum(cnt, 1.0)
    new_n = s_ref[0, 0] * fn1_ref[...] + s_ref[0, 1] * segm
    fn3 = jnp.maximum(_dot(new_n, w_ref[...]) + b_ref[...], 0.0)
    fn3_ref[...] = fn3
    fn3s_ref[...] = fn3 * s_ref[0, 2]

  return pl.pallas_call(
      body,
      grid=(grid,),
      in_specs=[
          pl.BlockSpec(memory_space=pltpu.SMEM),
          pl.BlockSpec((_BN, d), lambda i: (i, 0)),
          pl.BlockSpec((_BN, d), lambda i: (i, 0)),
          pl.BlockSpec((_BN, d), lambda i, nb=nblocks: (nb + i, 0)),
          pl.BlockSpec((_BN, 128), lambda i: (i, 0)),
          pl.BlockSpec((_BN, 128), lambda i, nb=nblocks: (nb + i, 0)),
          pl.BlockSpec((d, d), lambda i: (0, 0)),
          pl.BlockSpec((1, d), lambda i: (0, 0)),
      ],
      out_specs=[pl.BlockSpec((_BN, d), lambda i: (i, 0))] * 2,
      out_shape=[jax.ShapeDtypeStruct((n, d), jnp.float32)] * 2,
  )(scal, fn1, sums, sums, cnts, cnts, w3n, b3n.reshape(1, d))


def _tc_node_final(scal, fn3, sums, cnts,
                   w1, b1, w2, b2, w3, b3, w4, b4, wo, bo):
  n, d = fn3.shape
  nc = wo.shape[1]
  grid = n // _BN
  nblocks = n // _BN

  def body(s_ref, fn3_ref, s0_ref, s1_ref, c0_ref, c1_ref,
           w1_ref, b1_ref, w2_ref, b2_ref, w3_ref, b3_ref, w4_ref, b4_ref,
           wo_ref, bo_ref, o_ref):
    cnt = c0_ref[:, 0:1] + c1_ref[:, 0:1]
    segm = (s0_ref[...] + s1_ref[...]) / jnp.maximum(cnt, 1.0)
    new_n = s_ref[0, 0] * fn3_ref[...] + s_ref[0, 1] * segm
    o_ref[...] = _mlp56(new_n, w1_ref[...], b1_ref[...], w2_ref[...],
                        b2_ref[...], w3_ref[...], b3_ref[...], w4_ref[...],
                        b4_ref[...], wo_ref[...], bo_ref[...])

  wspec = pl.BlockSpec((d, d), lambda i: (0, 0))
  bspec = pl.BlockSpec((1, d), lambda i: (0, 0))
  return pl.pallas_call(
      body,
      grid=(grid,),
      in_specs=[
          pl.BlockSpec(memory_space=pltpu.SMEM),
          pl.BlockSpec((_BN, d), lambda i: (i, 0)),
          pl.BlockSpec((_BN, d), lambda i: (i, 0)),
          pl.BlockSpec((_BN, d), lambda i, nb=nblocks: (nb + i, 0)),
          pl.BlockSpec((_BN, 128), lambda i: (i, 0)),
          pl.BlockSpec((_BN, 128), lambda i, nb=nblocks: (nb + i, 0)),
          wspec, bspec, wspec, bspec, wspec, bspec, wspec, bspec,
          pl.BlockSpec((d, nc), lambda i: (0, 0)),
          pl.BlockSpec((1, nc), lambda i: (0, 0)),
      ],
      out_specs=pl.BlockSpec((_BN, nc), lambda i: (i, 0)),
      out_shape=jax.ShapeDtypeStruct((n, nc), jnp.float32),
  )(scal, fn3, sums, sums, cnts, cnts, w1, b1.reshape(1, d), w2,
    b2.reshape(1, d), w3, b3.reshape(1, d), w4, b4.reshape(1, d),
    wo, bo.reshape(1, nc))


def kernel(h, edge_index, W_embed, b_embed, W1n, b1n, W1e, b1e,
           w2_nn, w2_ne, w2_en, w2_ee, W3n, b3n, W3e, b3e,
           w4_nn, w4_ne, w4_en, w4_ee, W5_1, b5_1, W5_2, b5_2,
           W5_3, b5_3, W5_4, b5_4, W5_out, b5_out):
  n = h.shape[0]
  src = edge_index[0]
  dst = edge_index[1]

  a2 = (0.5 * w2_en).reshape(1, 1)
  h0, fn1, fn1s = _tc_embed(h, W_embed, b_embed, W1n, b1n, a2)

  cnts = _sc_counts(dst, n)
  ga1, gb1 = _sc_gather_pair(h0, src, dst)
  fe1 = _tc_mm1(ga1, gb1, 0.5 * W1e, b1e)
  sums1 = _sc_segsum(fe1, dst, n)

  scal_mid = jnp.concatenate([w2_nn, w2_ne, 0.5 * w4_en]).reshape(1, 3)
  fn3, fn3s = _tc_node_mid(scal_mid, fn1, sums1, cnts, W3n, b3n)

  ga2, gb2 = _sc_gather_pair(fn1s, src, dst)
  fe3 = _tc_mm2(w2_ee.reshape(1, 1), ga2, gb2, fe1, W3e, b3e)
  sums2 = _sc_segsum(fe3, dst, n)

  ga3, gb3 = _sc_gather_pair(fn3s, src, dst)
  out_e = _tc_mlp56_edges(w4_ee.reshape(1, 1), ga3, gb3, fe3,
                          W5_1, b5_1, W5_2, b5_2, W5_3, b5_3, W5_4, b5_4,
                          W5_out, b5_out)

  scal_fin = jnp.concatenate([w4_nn, w4_ne]).reshape(1, 2)
  out_n = _tc_node_final(scal_fin, fn3, sums2, cnts,
                         W5_1, b5_1, W5_2, b5_2, W5_3, b5_3, W5_4, b5_4,
                         W5_out, b5_out)
  return out_n, out_e


# speedup vs baseline: 3.5991x; 1.4682x over previous
import functools

import jax
import jax.numpy as jnp
from jax import lax
from jax.experimental import pallas as pl
from jax.experimental.pallas import tpu as pltpu
from jax.experimental.pallas import tpu_sc as plsc

_NC = 2
_NS = 16
_NW = _NC * _NS

_BE = 4000
_BN = 2000
_C = 80


def _sc_gather_pair(table, src, dst):
  n, d = table.shape
  e = src.shape[0]
  epw = e // _NW
  cg = 128
  steps = epw // cg
  tailc = epw - steps * cg
  pairs = steps // 2
  mesh = plsc.VectorSubcoreMesh(core_axis_name="c", subcore_axis_name="s")

  @functools.partial(
      pl.kernel,
      out_type=(jax.ShapeDtypeStruct((e, d), jnp.float32),
                jax.ShapeDtypeStruct((e, d), jnp.float32)),
      mesh=mesh,
      scratch_types=[
          pltpu.VMEM((cg,), jnp.int32),
          pltpu.VMEM((cg,), jnp.int32),
          pltpu.VMEM((cg,), jnp.int32),
          pltpu.VMEM((cg,), jnp.int32),
          pltpu.VMEM((cg, d), jnp.float32),
          pltpu.VMEM((cg, d), jnp.float32),
          pltpu.VMEM((cg, d), jnp.float32),
          pltpu.VMEM((cg, d), jnp.float32),
          pltpu.SemaphoreType.DMA,
          pltpu.SemaphoreType.DMA,
          pltpu.SemaphoreType.DMA,
          pltpu.SemaphoreType.DMA,
      ],
  )
  def k(table_h, src_h, dst_h, oa_h, ob_h,
        ia0, ib0, ia1, ib1, ra0, rb0, ra1, rb1, sa0, sb0, sa1, sb1):
    wid = lax.axis_index("s") * _NC + lax.axis_index("c")
    base = wid * epw

    def start(off, ia, ib, ra, rb, sa, sb):
      pltpu.sync_copy(src_h.at[pl.ds(off, cg)], ia)
      pltpu.sync_copy(dst_h.at[pl.ds(off, cg)], ib)
      pltpu.async_copy(table_h.at[ia], ra, sa)
      pltpu.async_copy(table_h.at[ib], rb, sb)

    def finish(off, ia, ib, ra, rb, sa, sb):
      pltpu.make_async_copy(table_h.at[ia], ra, sa).wait()
      pltpu.make_async_copy(table_h.at[ib], rb, sb).wait()
      pltpu.sync_copy(ra, oa_h.at[pl.ds(off, cg)])
      pltpu.sync_copy(rb, ob_h.at[pl.ds(off, cg)])

    start(base, ia0, ib0, ra0, rb0, sa0, sb0)

    def pair(j, carry):
      off0 = base + (2 * j) * cg
      off1 = off0 + cg
      start(off1, ia1, ib1, ra1, rb1, sa1, sb1)
      finish(off0, ia0, ib0, ra0, rb0, sa0, sb0)

      @pl.when(j < pairs - 1)
      def _start_next():
        start(off1 + cg, ia0, ib0, ra0, rb0, sa0, sb0)

      finish(off1, ia1, ib1, ra1, rb1, sa1, sb1)
      return carry

    lax.fori_loop(0, pairs, pair, 0)

    toff = base + steps * cg
    pltpu.sync_copy(src_h.at[pl.ds(toff, tailc)], ia0.at[pl.ds(0, tailc)])
    pltpu.sync_copy(dst_h.at[pl.ds(toff, tailc)], ib0.at[pl.ds(0, tailc)])
    cpa = pltpu.async_copy(table_h.at[ia0.at[pl.ds(0, tailc)]],
                           ra0.at[pl.ds(0, tailc)], sa0)
    cpb = pltpu.async_copy(table_h.at[ib0.at[pl.ds(0, tailc)]],
                           rb0.at[pl.ds(0, tailc)], sb0)
    cpa.wait()
    cpb.wait()
    pltpu.sync_copy(ra0.at[pl.ds(0, tailc)], oa_h.at[pl.ds(toff, tailc)])
    pltpu.sync_copy(rb0.at[pl.ds(0, tailc)], ob_h.at[pl.ds(toff, tailc)])

  return k(table, src, dst)


def _sc_segsum(feat, dst, n):
  e, d = feat.shape
  epw = e // _NW
  cg = 128
  steps = epw // cg
  tailc = epw - steps * cg
  pairs = steps // 2
  rps = (n // (_NS * 8)) * 8
  tail = n - rps * _NS
  zrows = rps // 13
  mesh = plsc.VectorSubcoreMesh(core_axis_name="c", subcore_axis_name="s")

  @functools.partial(
      pl.kernel,
      out_type=jax.ShapeDtypeStruct((2 * n, d), jnp.float32),
      mesh=mesh,
      scratch_types=[
          pltpu.VMEM((cg,), jnp.int32),
          pltpu.VMEM((cg,), jnp.int32),
          pltpu.VMEM((cg, d), jnp.float32),
          pltpu.VMEM((cg, d), jnp.float32),
          pltpu.VMEM((zrows, d), jnp.float32),
          pltpu.VMEM((16, d), jnp.float32),
          pltpu.VMEM((16,), jnp.int32),
          pltpu.VMEM_SHARED((n, d), jnp.float32),
          pltpu.SemaphoreType.DMA,
          pltpu.SemaphoreType.DMA,
      ],
  )
  def k(feat_h, dst_h, sums_h, idx0, idx1, rows0, rows1, zbuf_v,
        rowst, idxt, acc_s, sm0, sm1):
    cid = lax.axis_index("c")
    sid = lax.axis_index("s")
    wid = sid * _NC + cid

    zero16 = jnp.zeros((16,), jnp.float32)

    def fill_z(r, carry):
      for kk in range(d // 16):
        zbuf_v[r, pl.ds(16 * kk, 16)] = zero16
      return carry

    lax.fori_loop(0, zrows, fill_z, 0)

    rbase = sid * rps
    for j in range(rps // zrows):
      pltpu.sync_copy(zbuf_v, acc_s.at[pl.ds(rbase + j * zrows, zrows)])

    @pl.when(sid == _NS - 1)
    def _zero_tail():
      pltpu.sync_copy(zbuf_v.at[pl.ds(0, tail)],
                      acc_s.at[pl.ds(rps * _NS, tail)])

    plsc.subcore_barrier()

    base = wid * epw

    def start(off, idx, rows, sm):
      pltpu.sync_copy(dst_h.at[pl.ds(off, cg)], idx)
      pltpu.async_copy(feat_h.at[pl.ds(off, cg)], rows, sm)

    def finish(off, idx, rows, sm):
      pltpu.make_async_copy(feat_h.at[pl.ds(off, cg)], rows, sm).wait()
      pltpu.sync_copy(rows, acc_s.at[idx], add=True)

    start(base, idx0, rows0, sm0)

    def pair(j, carry):
      off0 = base + (2 * j) * cg
      off1 = off0 + cg
      start(off1, idx1, rows1, sm1)
      finish(off0, idx0, rows0, sm0)

      @pl.when(j < pairs - 1)
      def _start_next():
        start(off1 + cg, idx0, rows0, sm0)

      finish(off1, idx1, rows1, sm1)
      return carry

    lax.fori_loop(0, pairs, pair, 0)

    toff = base + steps * cg
    pltpu.sync_copy(dst_h.at[pl.ds(toff, tailc)], idxt)
    pltpu.sync_copy(feat_h.at[pl.ds(toff, tailc)], rowst)
    pltpu.sync_copy(rowst, acc_s.at[idxt], add=True)

    plsc.subcore_barrier()

    obase = cid * n + rbase
    pltpu.sync_copy(acc_s.at[pl.ds(rbase, rps)], sums_h.at[pl.ds(obase, rps)])

    @pl.when(sid == _NS - 1)
    def _write_tail():
      tbase = rps * _NS
      pltpu.sync_copy(acc_s.at[pl.ds(tbase, tail)],
                      sums_h.at[pl.ds(cid * n + tbase, tail)])

  return k(feat, dst)


def _sc_counts(dst, n):
  e = dst.shape[0]
  epw = e // _NW
  steps = epw // _C
  d = 128
  rps = (n // (_NS * 8)) * 8
  tail = n - rps * _NS
  zrows = rps // 3
  mesh = plsc.VectorSubcoreMesh(core_axis_name="c", subcore_axis_name="s")

  @functools.partial(
      pl.kernel,
      out_type=jax.ShapeDtypeStruct((2 * n, d), jnp.float32),
      mesh=mesh,
      scratch_types=[
          pltpu.VMEM((_C,), jnp.int32),
          pltpu.VMEM((_C, d), jnp.float32),
          pltpu.VMEM((zrows, d), jnp.float32),
          pltpu.VMEM_SHARED((n, d), jnp.float32),
      ],
  )
  def k(dst_h, cnts_h, idx_v, ones_v, zbuf_v, cnt_s):
    cid = lax.axis_index("c")
    sid = lax.axis_index("s")
    wid = sid * _NC + cid

    zero16 = jnp.zeros((16,), jnp.float32)
    one16 = jnp.ones((16,), jnp.float32)

    def fill_z(r, carry):
      for kk in range(d // 16):
        zbuf_v[r, pl.ds(16 * kk, 16)] = zero16
      return carry

    lax.fori_loop(0, zrows, fill_z, 0)

    def fill_o(r, carry):
      for kk in range(d // 16):
        ones_v[r, pl.ds(16 * kk, 16)] = one16
      return carry

    lax.fori_loop(0, _C, fill_o, 0)

    rbase = sid * rps
    for j in range(rps // zrows):
      pltpu.sync_copy(zbuf_v, cnt_s.at[pl.ds(rbase + j * zrows, zrows)])

    @pl.when(sid == _NS - 1)
    def _zero_tail():
      pltpu.sync_copy(zbuf_v.at[pl.ds(0, tail)],
                      cnt_s.at[pl.ds(rps * _NS, tail)])

    plsc.subcore_barrier()

    base = wid * epw

    def step(i, carry):
      off = base + i * _C
      pltpu.sync_copy(dst_h.at[pl.ds(off, _C)], idx_v)
      pltpu.sync_copy(ones_v, cnt_s.at[idx_v], add=True)
      return carry

    lax.fori_loop(0, steps, step, 0)

    plsc.subcore_barrier()

    obase = cid * n + rbase
    pltpu.sync_copy(cnt_s.at[pl.ds(rbase, rps)], cnts_h.at[pl.ds(obase, rps)])

    @pl.when(sid == _NS - 1)
    def _write_tail():
      tbase = rps * _NS
      pltpu.sync_copy(cnt_s.at[pl.ds(tbase, tail)],
                      cnts_h.at[pl.ds(cid * n + tbase, tail)])

  return k(dst)


def _dot(a, b):
  return jnp.dot(a, b, preferred_element_type=jnp.float32)


def _tc_embed(h, we, be, w1n, b1n, a_s):
  n, f = h.shape
  d = we.shape[1]
  grid = n // _BN

  def body(a_ref, h_ref, we_ref, be_ref, w1_ref, b1_ref,
           h0_ref, fn1_ref, fn1s_ref):
    h0 = _dot(h_ref[...], we_ref[...]) + be_ref[...]
    h0_ref[...] = h0
    fn1 = jnp.maximum(_dot(h0, w1_ref[...]) + b1_ref[...], 0.0)
    fn1_ref[...] = fn1
    fn1s_ref[...] = fn1 * a_ref[0, 0]

  return pl.pallas_call(
      body,
      grid=(grid,),
      in_specs=[
          pl.BlockSpec(memory_space=pltpu.SMEM),
          pl.BlockSpec((_BN, f), lambda i: (i, 0)),
          pl.BlockSpec((f, d), lambda i: (0, 0)),
          pl.BlockSpec((1, d), lambda i: (0, 0)),
          pl.BlockSpec((d, d), lambda i: (0, 0)),
          pl.BlockSpec((1, d), lambda i: (0, 0)),
      ],
      out_specs=[pl.BlockSpec((_BN, d), lambda i: (i, 0))] * 3,
      out_shape=[jax.ShapeDtypeStruct((n, d), jnp.float32)] * 3,
  )(a_s, h, we, be.reshape(1, d), w1n, b1n.reshape(1, d))


def _tc_mm1(ga, gb, w, b):
  e, d = ga.shape
  grid = e // _BE

  def body(ga_ref, gb_ref, w_ref, b_ref, o_ref):
    x = ga_ref[...] + gb_ref[...]
    o_ref[...] = jnp.maximum(_dot(x, w_ref[...]) + b_ref[...], 0.0)

  return pl.pallas_call(
      body,
      grid=(grid,),
      in_specs=[
          pl.BlockSpec((_BE, d), lambda i: (i, 0)),
          pl.BlockSpec((_BE, d), lambda i: (i, 0)),
          pl.BlockSpec((d, d), lambda i: (0, 0)),
          pl.BlockSpec((1, d), lambda i: (0, 0)),
      ],
      out_specs=pl.BlockSpec((_BE, d), lambda i: (i, 0)),
      out_shape=jax.ShapeDtypeStruct((e, d), jnp.float32),
  )(ga, gb, w, b.reshape(1, d))


def _tc_mm2_mlp56(s2, ga2, gb2, fe1, ga3, gb3, w3e, b3e,
                  w1, b1, w2, b2, w3, b3, w4, b4, wo, bo):
  e, d = ga2.shape
  nc = wo.shape[1]
  grid = e // _BE

  def body(s_ref, ga2_ref, gb2_ref, fe1_ref, ga3_ref, gb3_ref,
           w3e_ref, b3e_ref,
           w1_ref, b1_ref, w2_ref, b2_ref, w3_ref, b3_ref, w4_ref, b4_ref,
           wo_ref, bo_ref, fe3_ref, o_ref):
    x = ga2_ref[...] + gb2_ref[...] + s_ref[0, 0] * fe1_ref[...]
    fe3 = jnp.maximum(_dot(x, w3e_ref[...]) + b3e_ref[...], 0.0)
    fe3_ref[...] = fe3
    y = ga3_ref[...] + gb3_ref[...] + s_ref[0, 1] * fe3
    o_ref[...] = _mlp56(y, w1_ref[...], b1_ref[...], w2_ref[...], b2_ref[...],
                        w3_ref[...], b3_ref[...], w4_ref[...], b4_ref[...],
                        wo_ref[...], bo_ref[...])

  espec = pl.BlockSpec((_BE, d), lambda i: (i, 0))
  wspec = pl.BlockSpec((d, d), lambda i: (0, 0))
  bspec = pl.BlockSpec((1, d), lambda i: (0, 0))
  return pl.pallas_call(
      body,
      grid=(grid,),
      in_specs=[
          pl.BlockSpec(memory_space=pltpu.SMEM),
          espec, espec, espec, espec, espec,
          wspec, bspec,
          wspec, bspec, wspec, bspec, wspec, bspec, wspec, bspec,
          pl.BlockSpec((d, nc), lambda i: (0, 0)),
          pl.BlockSpec((1, nc), lambda i: (0, 0)),
      ],
      out_specs=[pl.BlockSpec((_BE, d), lambda i: (i, 0)),
                 pl.BlockSpec((_BE, nc), lambda i: (i, 0))],
      out_shape=[jax.ShapeDtypeStruct((e, d), jnp.float32),
                 jax.ShapeDtypeStruct((e, nc), jnp.float32)],
  )(s2, ga2, gb2, fe1, ga3, gb3, w3e, b3e.reshape(1, d),
    w1, b1.reshape(1, d), w2, b2.reshape(1, d), w3, b3.reshape(1, d),
    w4, b4.reshape(1, d), wo, bo.reshape(1, nc))


def _mlp56(x, w1, b1, w2, b2, w3, b3, w4, b4, wo, bo):
  x = jnp.maximum(_dot(x, w1) + b1, 0.0)
  x = jnp.maximum(_dot(x, w2) + b2, 0.0)
  x = jnp.maximum(_dot(x, w3) + b3, 0.0)
  x = jnp.maximum(_dot(x, w4) + b4, 0.0)
  return _dot(x, wo) + bo


def _tc_mlp56_edges(s, ga, gb, fe, w1, b1, w2, b2, w3, b3, w4, b4, wo, bo):
  e, d = ga.shape
  nc = wo.shape[1]
  grid = e // _BE

  def body(s_ref, ga_ref, gb_ref, fe_ref,
           w1_ref, b1_ref, w2_ref, b2_ref, w3_ref, b3_ref, w4_ref, b4_ref,
           wo_ref, bo_ref, o_ref):
    x = ga_ref[...] + gb_ref[...] + s_ref[0, 0] * fe_ref[...]
    o_ref[...] = _mlp56(x, w1_ref[...], b1_ref[...], w2_ref[...], b2_ref[...],
                        w3_ref[...], b3_ref[...], w4_ref[...], b4_ref[...],
                        wo_ref[...], bo_ref[...])

  wspec = pl.BlockSpec((d, d), lambda i: (0, 0))
  bspec = pl.BlockSpec((1, d), lambda i: (0, 0))
  espec = pl.BlockSpec((_BE, d), lambda i: (i, 0))
  return pl.pallas_call(
      body,
      grid=(grid,),
      in_specs=[
          pl.BlockSpec(memory_space=pltpu.SMEM),
          espec, espec, espec,
          wspec, bspec, wspec, bspec, wspec, bspec, wspec, bspec,
          pl.BlockSpec((d, nc), lambda i: (0, 0)),
          pl.BlockSpec((1, nc), lambda i: (0, 0)),
      ],
      out_specs=pl.BlockSpec((_BE, nc), lambda i: (i, 0)),
      out_shape=jax.ShapeDtypeStruct((e, nc), jnp.float32),
  )(s, ga, gb, fe, w1, b1.reshape(1, d), w2, b2.reshape(1, d),
    w3, b3.reshape(1, d), w4, b4.reshape(1, d), wo, bo.reshape(1, nc))


def _tc_node_mid(scal, fn1, sums, cnts, w3n, b3n):
  n, d = fn1.shape
  grid = n // _BN
  nblocks = n // _BN

  def body(s_ref, fn1_ref, s0_ref, s1_ref, c0_ref, c1_ref, w_ref, b_ref,
           fn3_ref, fn3s_ref):
    cnt = c0_ref[:, 0:1] + c1_ref[:, 0:1]
    segm = (s0_ref[...] + s1_ref[...]) / jnp.maximum(cnt, 1.0)
    new_n = s_ref[0, 0] * fn1_ref[...] + s_ref[0, 1] * segm
    fn3 = jnp.maximum(_dot(new_n, w_ref[...]) + b_ref[...], 0.0)
    fn3_ref[...] = fn3
    fn3s_ref[...] = fn3 * s_ref[0, 2]

  return pl.pallas_call(
      body,
      grid=(grid,),
      in_specs=[
          pl.BlockSpec(memory_space=pltpu.SMEM),
          pl.BlockSpec((_BN, d), lambda i: (i, 0)),
          pl.BlockSpec((_BN, d), lambda i: (i, 0)),
          pl.BlockSpec((_BN, d), lambda i, nb=nblocks: (nb + i, 0)),
          pl.BlockSpec((_BN, 128), lambda i: (i, 0)),
          pl.BlockSpec((_BN, 128), lambda i, nb=nblocks: (nb + i, 0)),
          pl.BlockSpec((d, d), lambda i: (0, 0)),
          pl.BlockSpec((1, d), lambda i: (0, 0)),
      ],
      out_specs=[pl.BlockSpec((_BN, d), lambda i: (i, 0))] * 2,
      out_shape=[jax.ShapeDtypeStruct((n, d), jnp.float32)] * 2,
  )(scal, fn1, sums, sums, cnts, cnts, w3n, b3n.reshape(1, d))


def _tc_node_final(scal, fn3, sums, cnts,
                   w1, b1, w2, b2, w3, b3, w4, b4, wo, bo):
  n, d = fn3.shape
  nc = wo.shape[1]
  grid = n // _BN
  nblocks = n // _BN

  def body(s_ref, fn3_ref, s0_ref, s1_ref, c0_ref, c1_ref,
           w1_ref, b1_ref, w2_ref, b2_ref, w3_ref, b3_ref, w4_ref, b4_ref,
           wo_ref, bo_ref, o_ref):
    cnt = c0_ref[:, 0:1] + c1_ref[:, 0:1]
    segm = (s0_ref[...] + s1_ref[...]) / jnp.maximum(cnt, 1.0)
    new_n = s_ref[0, 0] * fn3_ref[...] + s_ref[0, 1] * segm
    o_ref[...] = _mlp56(new_n, w1_ref[...], b1_ref[...], w2_ref[...],
                        b2_ref[...], w3_ref[...], b3_ref[...], w4_ref[...],
                        b4_ref[...], wo_ref[...], bo_ref[...])

  wspec = pl.BlockSpec((d, d), lambda i: (0, 0))
  bspec = pl.BlockSpec((1, d), lambda i: (0, 0))
  return pl.pallas_call(
      body,
      grid=(grid,),
      in_specs=[
          pl.BlockSpec(memory_space=pltpu.SMEM),
          pl.BlockSpec((_BN, d), lambda i: (i, 0)),
          pl.BlockSpec((_BN, d), lambda i: (i, 0)),
          pl.BlockSpec((_BN, d), lambda i, nb=nblocks: (nb + i, 0)),
          pl.BlockSpec((_BN, 128), lambda i: (i, 0)),
          pl.BlockSpec((_BN, 128), lambda i, nb=nblocks: (nb + i, 0)),
          wspec, bspec, wspec, bspec, wspec, bspec, wspec, bspec,
          pl.BlockSpec((d, nc), lambda i: (0, 0)),
          pl.BlockSpec((1, nc), lambda i: (0, 0)),
      ],
      out_specs=pl.BlockSpec((_BN, nc), lambda i: (i, 0)),
      out_shape=jax.ShapeDtypeStruct((n, nc), jnp.float32),
  )(scal, fn3, sums, sums, cnts, cnts, w1, b1.reshape(1, d), w2,
    b2.reshape(1, d), w3, b3.reshape(1, d), w4, b4.reshape(1, d),
    wo, bo.reshape(1, nc))


def kernel(h, edge_index, W_embed, b_embed, W1n, b1n, W1e, b1e,
           w2_nn, w2_ne, w2_en, w2_ee, W3n, b3n, W3e, b3e,
           w4_nn, w4_ne, w4_en, w4_ee, W5_1, b5_1, W5_2, b5_2,
           W5_3, b5_3, W5_4, b5_4, W5_out, b5_out):
  n = h.shape[0]
  src = edge_index[0]
  dst = edge_index[1]

  a2 = (0.5 * w2_en).reshape(1, 1)
  h0, fn1, fn1s = _tc_embed(h, W_embed, b_embed, W1n, b1n, a2)

  cnts = _sc_counts(dst, n)
  ga1, gb1 = _sc_gather_pair(h0, src, dst)
  fe1 = _tc_mm1(ga1, gb1, 0.5 * W1e, b1e)
  sums1 = _sc_segsum(fe1, dst, n)

  scal_mid = jnp.concatenate([w2_nn, w2_ne, 0.5 * w4_en]).reshape(1, 3)
  fn3, fn3s = _tc_node_mid(scal_mid, fn1, sums1, cnts, W3n, b3n)

  ga2, gb2 = _sc_gather_pair(fn1s, src, dst)
  ga3, gb3 = _sc_gather_pair(fn3s, src, dst)
  s2 = jnp.concatenate([w2_ee, w4_ee]).reshape(1, 2)
  fe3, out_e = _tc_mm2_mlp56(s2, ga2, gb2, fe1, ga3, gb3, W3e, b3e,
                             W5_1, b5_1, W5_2, b5_2, W5_3, b5_3, W5_4, b5_4,
                             W5_out, b5_out)
  sums2 = _sc_segsum(fe3, dst, n)

  scal_fin = jnp.concatenate([w4_nn, w4_ne]).reshape(1, 2)
  out_n = _tc_node_final(scal_fin, fn3, sums2, cnts,
                         W5_1, b5_1, W5_2, b5_2, W5_3, b5_3, W5_4, b5_4,
                         W5_out, b5_out)
  return out_n, out_e


# speedup vs baseline: 3.8563x; 1.0714x over previous
import functools

import jax
import jax.numpy as jnp
from jax import lax
from jax.experimental import pallas as pl
from jax.experimental.pallas import tpu as pltpu
from jax.experimental.pallas import tpu_sc as plsc

_NC = 2
_NS = 16
_NW = _NC * _NS

_BE = 4000
_BN = 2000
_C = 80


def _sc_gather_pair(table, src2, dst2, xsrc1, xdst1):
  n, d = table.shape
  nch, cg = src2.shape
  e = nch * cg
  cpw = -(-nch // _NW)
  cpw += (-cpw) % 8
  last = _NW - 1
  nlast = ((nch - last * cpw) // 8) * 8
  rem = nch - last * cpw - nlast
  mesh = plsc.VectorSubcoreMesh(core_axis_name="c", subcore_axis_name="s")

  @functools.partial(
      pl.kernel,
      out_type=(jax.ShapeDtypeStruct((e, d), jnp.float32),
                jax.ShapeDtypeStruct((e, d), jnp.float32)),
      mesh=mesh,
      scratch_types=[
          pltpu.VMEM((cpw, cg), jnp.int32),
          pltpu.VMEM((cpw, cg), jnp.int32),
          pltpu.VMEM((cg,), jnp.int32),
          pltpu.VMEM((cg,), jnp.int32),
          pltpu.VMEM((cg, d), jnp.float32),
          pltpu.VMEM((cg, d), jnp.float32),
          pltpu.VMEM((cg, d), jnp.float32),
          pltpu.VMEM((cg, d), jnp.float32),
          pltpu.SemaphoreType.DMA,
          pltpu.SemaphoreType.DMA,
          pltpu.SemaphoreType.DMA,
          pltpu.SemaphoreType.DMA,
      ],
  )
  def k(table_h, src_h, dst_h, xsrc_h, xdst_h, oa_h, ob_h,
        srcb, dstb, xsrc, xdst, ra0, rb0, ra1, rb1, sa0, sb0, sa1, sb1):
    wid = lax.axis_index("s") * _NC + lax.axis_index("c")
    crow = wid * cpw
    ebase = crow * cg
    pairs = jnp.where(wid == last, nlast // 2, cpw // 2)

    @pl.when(wid < last)
    def _prefetch_full():
      pltpu.sync_copy(src_h.at[pl.ds(crow, cpw)], srcb)
      pltpu.sync_copy(dst_h.at[pl.ds(crow, cpw)], dstb)

    @pl.when(wid == last)
    def _prefetch_part():
      pltpu.sync_copy(src_h.at[pl.ds(crow, nlast)], srcb.at[pl.ds(0, nlast)])
      pltpu.sync_copy(dst_h.at[pl.ds(crow, nlast)], dstb.at[pl.ds(0, nlast)])

    def start(i, ra, rb, sa, sb):
      pltpu.async_copy(table_h.at[srcb.at[i]], ra, sa)
      pltpu.async_copy(table_h.at[dstb.at[i]], rb, sb)

    def finish(i, ra, rb, sa, sb):
      off = ebase + i * cg
      pltpu.make_async_copy(table_h.at[srcb.at[i]], ra, sa).wait()
      pltpu.make_async_copy(table_h.at[dstb.at[i]], rb, sb).wait()
      pltpu.sync_copy(ra, oa_h.at[pl.ds(off, cg)])
      pltpu.sync_copy(rb, ob_h.at[pl.ds(off, cg)])

    start(0, ra0, rb0, sa0, sb0)

    def pair(j, carry):
      i0 = 2 * j
      start(i0 + 1, ra1, rb1, sa1, sb1)
      finish(i0, ra0, rb0, sa0, sb0)

      @pl.when(j < pairs - 1)
      def _start_next():
        start(i0 + 2, ra0, rb0, sa0, sb0)

      finish(i0 + 1, ra1, rb1, sa1, sb1)
      return carry

    lax.fori_loop(0, pairs, pair, 0)

    @pl.when(wid < rem)
    def _extra():
      xoff = (nch - rem + wid) * cg
      pltpu.sync_copy(xsrc_h.at[pl.ds(wid * cg, cg)], xsrc)
      pltpu.sync_copy(xdst_h.at[pl.ds(wid * cg, cg)], xdst)
      cpa = pltpu.async_copy(table_h.at[xsrc], ra0, sa0)
      cpb = pltpu.async_copy(table_h.at[xdst], rb0, sb0)
      cpa.wait()
      cpb.wait()
      pltpu.sync_copy(ra0, oa_h.at[pl.ds(xoff, cg)])
      pltpu.sync_copy(rb0, ob_h.at[pl.ds(xoff, cg)])

  return k(table, src2, dst2, xsrc1, xdst1)


def _sc_segsum(feat, dst2, xdst1, n):
  e, d = feat.shape
  nch, cg = dst2.shape
  cpw = -(-nch // _NW)
  cpw += (-cpw) % 8
  last = _NW - 1
  nlast = ((nch - last * cpw) // 8) * 8
  rem = nch - last * cpw - nlast
  rps = (n // (_NS * 8)) * 8
  ztail = n - rps * _NS
  zrows = 16
  mesh = plsc.VectorSubcoreMesh(core_axis_name="c", subcore_axis_name="s")

  @functools.partial(
      pl.kernel,
      out_type=jax.ShapeDtypeStruct((2 * n, d), jnp.float32),
      mesh=mesh,
      scratch_types=[
          pltpu.VMEM((cpw, cg), jnp.int32),
          pltpu.VMEM((cg,), jnp.int32),
          pltpu.VMEM((cg, d), jnp.float32),
          pltpu.VMEM((cg, d), jnp.float32),
          pltpu.VMEM((zrows, d), jnp.float32),
          pltpu.VMEM_SHARED((n, d), jnp.float32),
          pltpu.SemaphoreType.DMA,
          pltpu.SemaphoreType.DMA,
      ],
  )
  def k(feat_h, dst_h, xdst_h, sums_h, dstb, xdst, rows0, rows1, zbuf_v,
        acc_s, sm0, sm1):
    cid = lax.axis_index("c")
    sid = lax.axis_index("s")
    wid = sid * _NC + cid
    crow = wid * cpw
    ebase = crow * cg
    pairs = jnp.where(wid == last, nlast // 2, cpw // 2)

    zero16 = jnp.zeros((16,), jnp.float32)

    def fill_z(r, carry):
      for kk in range(d // 16):
        zbuf_v[r, pl.ds(16 * kk, 16)] = zero16
      return carry

    lax.fori_loop(0, zrows, fill_z, 0)

    rbase = sid * rps
    for j in range(rps // zrows):
      pltpu.sync_copy(zbuf_v, acc_s.at[pl.ds(rbase + j * zrows, zrows)])

    @pl.when(sid == _NS - 1)
    def _zero_tail():
      pltpu.sync_copy(zbuf_v.at[pl.ds(0, ztail)],
                      acc_s.at[pl.ds(rps * _NS, ztail)])

    @pl.when(wid < last)
    def _prefetch_full():
      pltpu.sync_copy(dst_h.at[pl.ds(crow, cpw)], dstb)

    @pl.when(wid == last)
    def _prefetch_part():
      pltpu.sync_copy(dst_h.at[pl.ds(crow, nlast)], dstb.at[pl.ds(0, nlast)])

    plsc.subcore_barrier()

    def start(i, rows, sm):
      pltpu.async_copy(feat_h.at[pl.ds(ebase + i * cg, cg)], rows, sm)

    def finish(i, rows, sm):
      pltpu.make_async_copy(feat_h.at[pl.ds(ebase + i * cg, cg)],
                            rows, sm).wait()
      pltpu.sync_copy(rows, acc_s.at[dstb.at[i]], add=True)

    start(0, rows0, sm0)

    def pair(j, carry):
      i0 = 2 * j
      start(i0 + 1, rows1, sm1)
      finish(i0, rows0, sm0)

      @pl.when(j < pairs - 1)
      def _start_next():
        start(i0 + 2, rows0, sm0)

      finish(i0 + 1, rows1, sm1)
      return carry

    lax.fori_loop(0, pairs, pair, 0)

    @pl.when(wid < rem)
    def _extra():
      xoff = (nch - rem + wid) * cg
      pltpu.sync_copy(xdst_h.at[pl.ds(wid * cg, cg)], xdst)
      pltpu.sync_copy(feat_h.at[pl.ds(xoff, cg)], rows0)
      pltpu.sync_copy(rows0, acc_s.at[xdst], add=True)

    plsc.subcore_barrier()

    obase = cid * n + rbase
    pltpu.sync_copy(acc_s.at[pl.ds(rbase, rps)], sums_h.at[pl.ds(obase, rps)])

    @pl.when(sid == _NS - 1)
    def _write_tail():
      tbase = rps * _NS
      pltpu.sync_copy(acc_s.at[pl.ds(tbase, ztail)],
                      sums_h.at[pl.ds(cid * n + tbase, ztail)])

  return k(feat, dst2, xdst1)


def _sc_counts(dst2, xdst1, n):
  nch, cg = dst2.shape
  w = 128
  cpw = -(-nch // _NW)
  cpw += (-cpw) % 8
  last = _NW - 1
  nlast = ((nch - last * cpw) // 8) * 8
  rem = nch - last * cpw - nlast
  rps = (n // (_NS * 8)) * 8
  ztail = n - rps * _NS
  zrows = 48
  mesh = plsc.VectorSubcoreMesh(core_axis_name="c", subcore_axis_name="s")

  @functools.partial(
      pl.kernel,
      out_type=jax.ShapeDtypeStruct((2 * n, w), jnp.float32),
      mesh=mesh,
      scratch_types=[
          pltpu.VMEM((cpw, cg), jnp.int32),
          pltpu.VMEM((cg,), jnp.int32),
          pltpu.VMEM((cg, w), jnp.float32),
          pltpu.VMEM((zrows, w), jnp.float32),
          pltpu.VMEM_SHARED((n, w), jnp.float32),
      ],
  )
  def k(dst_h, xdst_h, cnts_h, dstb, xdst, ones_v, zbuf_v, cnt_s):
    cid = lax.axis_index("c")
    sid = lax.axis_index("s")
    wid = sid * _NC + cid
    crow = wid * cpw

    zero16 = jnp.zeros((16,), jnp.float32)
    one16 = jnp.ones((16,), jnp.float32)

    def fill_z(r, carry):
      for kk in range(w // 16):
        zbuf_v[r, pl.ds(16 * kk, 16)] = zero16
      return carry

    lax.fori_loop(0, zrows, fill_z, 0)

    def fill_o(r, carry):
      for kk in range(w // 16):
        ones_v[r, pl.ds(16 * kk, 16)] = one16
      return carry

    lax.fori_loop(0, cg, fill_o, 0)

    rbase = sid * rps
    for j in range(rps // zrows):
      pltpu.sync_copy(zbuf_v, cnt_s.at[pl.ds(rbase + j * zrows, zrows)])

    @pl.when(sid == _NS - 1)
    def _zero_tail():
      pltpu.sync_copy(zbuf_v.at[pl.ds(0, ztail)],
                      cnt_s.at[pl.ds(rps * _NS, ztail)])

    nloc = jnp.where(wid == last, nlast, cpw)

    @pl.when(wid < last)
    def _prefetch_full():
      pltpu.sync_copy(dst_h.at[pl.ds(crow, cpw)], dstb)

    @pl.when(wid == last)
    def _prefetch_part():
      pltpu.sync_copy(dst_h.at[pl.ds(crow, nlast)], dstb.at[pl.ds(0, nlast)])

    plsc.subcore_barrier()

    def step(i, carry):
      pltpu.sync_copy(ones_v, cnt_s.at[dstb.at[i]], add=True)
      return carry

    lax.fori_loop(0, nloc, step, 0)

    @pl.when(wid < rem)
    def _extra():
      pltpu.sync_copy(xdst_h.at[pl.ds(wid * cg, cg)], xdst)
      pltpu.sync_copy(ones_v, cnt_s.at[xdst], add=True)

    plsc.subcore_barrier()

    obase = cid * n + rbase
    pltpu.sync_copy(cnt_s.at[pl.ds(rbase, rps)], cnts_h.at[pl.ds(obase, rps)])

    @pl.when(sid == _NS - 1)
    def _write_tail():
      tbase = rps * _NS
      pltpu.sync_copy(cnt_s.at[pl.ds(tbase, ztail)],
                      cnts_h.at[pl.ds(cid * n + tbase, ztail)])

  return k(dst2, xdst1)


def _dot(a, b):
  return jnp.dot(a, b, preferred_element_type=jnp.float32)


def _tc_embed(h, we, be, w1n, b1n, a_s):
  n, f = h.shape
  d = we.shape[1]
  grid = n // _BN

  def body(a_ref, h_ref, we_ref, be_ref, w1_ref, b1_ref,
           h0_ref, fn1_ref, fn1s_ref):
    h0 = _dot(h_ref[...], we_ref[...]) + be_ref[...]
    h0_ref[...] = h0
    fn1 = jnp.maximum(_dot(h0, w1_ref[...]) + b1_ref[...], 0.0)
    fn1_ref[...] = fn1
    fn1s_ref[...] = fn1 * a_ref[0, 0]

  return pl.pallas_call(
      body,
      grid=(grid,),
      in_specs=[
          pl.BlockSpec(memory_space=pltpu.SMEM),
          pl.BlockSpec((_BN, f), lambda i: (i, 0)),
          pl.BlockSpec((f, d), lambda i: (0, 0)),
          pl.BlockSpec((1, d), lambda i: (0, 0)),
          pl.BlockSpec((d, d), lambda i: (0, 0)),
          pl.BlockSpec((1, d), lambda i: (0, 0)),
      ],
      out_specs=[pl.BlockSpec((_BN, d), lambda i: (i, 0))] * 3,
      out_shape=[jax.ShapeDtypeStruct((n, d), jnp.float32)] * 3,
  )(a_s, h, we, be.reshape(1, d), w1n, b1n.reshape(1, d))


def _tc_mm1(ga, gb, w, b):
  e, d = ga.shape
  grid = e // _BE

  def body(ga_ref, gb_ref, w_ref, b_ref, o_ref):
    x = ga_ref[...] + gb_ref[...]
    o_ref[...] = jnp.maximum(_dot(x, w_ref[...]) + b_ref[...], 0.0)

  return pl.pallas_call(
      body,
      grid=(grid,),
      in_specs=[
          pl.BlockSpec((_BE, d), lambda i: (i, 0)),
          pl.BlockSpec((_BE, d), lambda i: (i, 0)),
          pl.BlockSpec((d, d), lambda i: (0, 0)),
          pl.BlockSpec((1, d), lambda i: (0, 0)),
      ],
      out_specs=pl.BlockSpec((_BE, d), lambda i: (i, 0)),
      out_shape=jax.ShapeDtypeStruct((e, d), jnp.float32),
  )(ga, gb, w, b.reshape(1, d))


def _tc_mm2_mlp56(s2, ga2, gb2, fe1, ga3, gb3, w3e, b3e,
                  w1, b1, w2, b2, w3, b3, w4, b4, wo, bo):
  e, d = ga2.shape
  nc = wo.shape[1]
  grid = e // _BE

  def body(s_ref, ga2_ref, gb2_ref, fe1_ref, ga3_ref, gb3_ref,
           w3e_ref, b3e_ref,
           w1_ref, b1_ref, w2_ref, b2_ref, w3_ref, b3_ref, w4_ref, b4_ref,
           wo_ref, bo_ref, fe3_ref, o_ref):
    x = ga2_ref[...] + gb2_ref[...] + s_ref[0, 0] * fe1_ref[...]
    fe3 = jnp.maximum(_dot(x, w3e_ref[...]) + b3e_ref[...], 0.0)
    fe3_ref[...] = fe3
    y = ga3_ref[...] + gb3_ref[...] + s_ref[0, 1] * fe3
    o_ref[...] = _mlp56(y, w1_ref[...], b1_ref[...], w2_ref[...], b2_ref[...],
                        w3_ref[...], b3_ref[...], w4_ref[...], b4_ref[...],
                        wo_ref[...], bo_ref[...])

  espec = pl.BlockSpec((_BE, d), lambda i: (i, 0))
  wspec = pl.BlockSpec((d, d), lambda i: (0, 0))
  bspec = pl.BlockSpec((1, d), lambda i: (0, 0))
  return pl.pallas_call(
      body,
      grid=(grid,),
      in_specs=[
          pl.BlockSpec(memory_space=pltpu.SMEM),
          espec, espec, espec, espec, espec,
          wspec, bspec,
          wspec, bspec, wspec, bspec, wspec, bspec, wspec, bspec,
          pl.BlockSpec((d, nc), lambda i: (0, 0)),
          pl.BlockSpec((1, nc), lambda i: (0, 0)),
      ],
      out_specs=[pl.BlockSpec((_BE, d), lambda i: (i, 0)),
                 pl.BlockSpec((_BE, nc), lambda i: (i, 0))],
      out_shape=[jax.ShapeDtypeStruct((e, d), jnp.float32),
                 jax.ShapeDtypeStruct((e, nc), jnp.float32)],
  )(s2, ga2, gb2, fe1, ga3, gb3, w3e, b3e.reshape(1, d),
    w1, b1.reshape(1, d), w2, b2.reshape(1, d), w3, b3.reshape(1, d),
    w4, b4.reshape(1, d), wo, bo.reshape(1, nc))


def _mlp56(x, w1, b1, w2, b2, w3, b3, w4, b4, wo, bo):
  x = jnp.maximum(_dot(x, w1) + b1, 0.0)
  x = jnp.maximum(_dot(x, w2) + b2, 0.0)
  x = jnp.maximum(_dot(x, w3) + b3, 0.0)
  x = jnp.maximum(_dot(x, w4) + b4, 0.0)
  return _dot(x, wo) + bo


def _tc_mlp56_edges(s, ga, gb, fe, w1, b1, w2, b2, w3, b3, w4, b4, wo, bo):
  e, d = ga.shape
  nc = wo.shape[1]
  grid = e // _BE

  def body(s_ref, ga_ref, gb_ref, fe_ref,
           w1_ref, b1_ref, w2_ref, b2_ref, w3_ref, b3_ref, w4_ref, b4_ref,
           wo_ref, bo_ref, o_ref):
    x = ga_ref[...] + gb_ref[...] + s_ref[0, 0] * fe_ref[...]
    o_ref[...] = _mlp56(x, w1_ref[...], b1_ref[...], w2_ref[...], b2_ref[...],
                        w3_ref[...], b3_ref[...], w4_ref[...], b4_ref[...],
                        wo_ref[...], bo_ref[...])

  wspec = pl.BlockSpec((d, d), lambda i: (0, 0))
  bspec = pl.BlockSpec((1, d), lambda i: (0, 0))
  espec = pl.BlockSpec((_BE, d), lambda i: (i, 0))
  return pl.pallas_call(
      body,
      grid=(grid,),
      in_specs=[
          pl.BlockSpec(memory_space=pltpu.SMEM),
          espec, espec, espec,
          wspec, bspec, wspec, bspec, wspec, bspec, wspec, bspec,
          pl.BlockSpec((d, nc), lambda i: (0, 0)),
          pl.BlockSpec((1, nc), lambda i: (0, 0)),
      ],
      out_specs=pl.BlockSpec((_BE, nc), lambda i: (i, 0)),
      out_shape=jax.ShapeDtypeStruct((e, nc), jnp.float32),
  )(s, ga, gb, fe, w1, b1.reshape(1, d), w2, b2.reshape(1, d),
    w3, b3.reshape(1, d), w4, b4.reshape(1, d), wo, bo.reshape(1, nc))


def _tc_node_mid(scal, fn1, sums, cnts, w3n, b3n):
  n, d = fn1.shape
  grid = n // _BN
  nblocks = n // _BN

  def body(s_ref, fn1_ref, s0_ref, s1_ref, c0_ref, c1_ref, w_ref, b_ref,
           fn3_ref, fn3s_ref):
    cnt = c0_ref[:, 0:1] + c1_ref[:, 0:1]
    segm = (s0_ref[...] + s1_ref[...]) / jnp.maximum(cnt, 1.0)
    new_n = s_ref[0, 0] * fn1_ref[...] + s_ref[0, 1] * segm
    fn3 = jnp.maximum(_dot(new_n, w_ref[...]) + b_ref[...], 0.0)
    fn3_ref[...] = fn3
    fn3s_ref[...] = fn3 * s_ref[0, 2]

  return pl.pallas_call(
      body,
      grid=(grid,),
      in_specs=[
          pl.BlockSpec(memory_space=pltpu.SMEM),
          pl.BlockSpec((_BN, d), lambda i: (i, 0)),
          pl.BlockSpec((_BN, d), lambda i: (i, 0)),
          pl.BlockSpec((_BN, d), lambda i, nb=nblocks: (nb + i, 0)),
          pl.BlockSpec((_BN, 128), lambda i: (i, 0)),
          pl.BlockSpec((_BN, 128), lambda i, nb=nblocks: (nb + i, 0)),
          pl.BlockSpec((d, d), lambda i: (0, 0)),
          pl.BlockSpec((1, d), lambda i: (0, 0)),
      ],
      out_specs=[pl.BlockSpec((_BN, d), lambda i: (i, 0))] * 2,
      out_shape=[jax.ShapeDtypeStruct((n, d), jnp.float32)] * 2,
  )(scal, fn1, sums, sums, cnts, cnts, w3n, b3n.reshape(1, d))


def _tc_node_final(scal, fn3, sums, cnts,
                   w1, b1, w2, b2, w3, b3, w4, b4, wo, bo):
  n, d = fn3.shape
  nc = wo.shape[1]
  grid = n // _BN
  nblocks = n // _BN

  def body(s_ref, fn3_ref, s0_ref, s1_ref, c0_ref, c1_ref,
           w1_ref, b1_ref, w2_ref, b2_ref, w3_ref, b3_ref, w4_ref, b4_ref,
           wo_ref, bo_ref, o_ref):
    cnt = c0_ref[:, 0:1] + c1_ref[:, 0:1]
    segm = (s0_ref[...] + s1_ref[...]) / jnp.maximum(cnt, 1.0)
    new_n = s_ref[0, 0] * fn3_ref[...] + s_ref[0, 1] * segm
    o_ref[...] = _mlp56(new_n, w1_ref[...], b1_ref[...], w2_ref[...],
                        b2_ref[...], w3_ref[...], b3_ref[...], w4_ref[...],
                        b4_ref[...], wo_ref[...], bo_ref[...])

  wspec = pl.BlockSpec((d, d), lambda i: (0, 0))
  bspec = pl.BlockSpec((1, d), lambda i: (0, 0))
  return pl.pallas_call(
      body,
      grid=(grid,),
      in_specs=[
          pl.BlockSpec(memory_space=pltpu.SMEM),
          pl.BlockSpec((_BN, d), lambda i: (i, 0)),
          pl.BlockSpec((_BN, d), lambda i: (i, 0)),
          pl.BlockSpec((_BN, d), lambda i, nb=nblocks: (nb + i, 0)),
          pl.BlockSpec((_BN, 128), lambda i: (i, 0)),
          pl.BlockSpec((_BN, 128), lambda i, nb=nblocks: (nb + i, 0)),
          wspec, bspec, wspec, bspec, wspec, bspec, wspec, bspec,
          pl.BlockSpec((d, nc), lambda i: (0, 0)),
          pl.BlockSpec((1, nc), lambda i: (0, 0)),
      ],
      out_specs=pl.BlockSpec((_BN, nc), lambda i: (i, 0)),
      out_shape=jax.ShapeDtypeStruct((n, nc), jnp.float32),
  )(scal, fn3, sums, sums, cnts, cnts, w1, b1.reshape(1, d), w2,
    b2.reshape(1, d), w3, b3.reshape(1, d), w4, b4.reshape(1, d),
    wo, bo.reshape(1, nc))


def kernel(h, edge_index, W_embed, b_embed, W1n, b1n, W1e, b1e,
           w2_nn, w2_ne, w2_en, w2_ee, W3n, b3n, W3e, b3e,
           w4_nn, w4_ne, w4_en, w4_ee, W5_1, b5_1, W5_2, b5_2,
           W5_3, b5_3, W5_4, b5_4, W5_out, b5_out):
  n = h.shape[0]
  src2 = edge_index[0].reshape(-1, 128)
  dst2 = edge_index[1].reshape(-1, 128)
  nrem = 512
  xs1 = edge_index[0, -nrem:]
  xd1 = edge_index[1, -nrem:]

  a2 = (0.5 * w2_en).reshape(1, 1)
  h0, fn1, fn1s = _tc_embed(h, W_embed, b_embed, W1n, b1n, a2)

  cnts = _sc_counts(dst2, xd1, n)
  ga1, gb1 = _sc_gather_pair(h0, src2, dst2, xs1, xd1)
  fe1 = _tc_mm1(ga1, gb1, 0.5 * W1e, b1e)
  sums1 = _sc_segsum(fe1, dst2, xd1, n)

  scal_mid = jnp.concatenate([w2_nn, w2_ne, 0.5 * w4_en]).reshape(1, 3)
  fn3, fn3s = _tc_node_mid(scal_mid, fn1, sums1, cnts, W3n, b3n)

  ga2, gb2 = _sc_gather_pair(fn1s, src2, dst2, xs1, xd1)
  ga3, gb3 = _sc_gather_pair(fn3s, src2, dst2, xs1, xd1)
  s2 = jnp.concatenate([w2_ee, w4_ee]).reshape(1, 2)
  fe3, out_e = _tc_mm2_mlp56(s2, ga2, gb2, fe1, ga3, gb3, W3e, b3e,
                             W5_1, b5_1, W5_2, b5_2, W5_3, b5_3, W5_4, b5_4,
                             W5_out, b5_out)
  sums2 = _sc_segsum(fe3, dst2, xd1, n)

  scal_fin = jnp.concatenate([w4_nn, w4_ne]).reshape(1, 2)
  out_n = _tc_node_final(scal_fin, fn3, sums2, cnts,
                         W5_1, b5_1, W5_2, b5_2, W5_3, b5_3, W5_4, b5_4,
                         W5_out, b5_out)
  return out_n, out_e


# speedup vs baseline: 4.6810x; 1.2139x over previous
import functools

import jax
import jax.numpy as jnp
from jax import lax
from jax.experimental import pallas as pl
from jax.experimental.pallas import tpu as pltpu
from jax.experimental.pallas import tpu_sc as plsc

_NC = 2
_NS = 16
_NW = _NC * _NS

_BE = 4000
_BN = 2000
_C = 80


def _sc_gather_pair(table, src2, dst2, xsrc1, xdst1):
  n, d = table.shape
  nch, cg = src2.shape
  e = nch * cg
  cpw = -(-nch // _NW)
  cpw += (-cpw) % 8
  last = _NW - 1
  nlast = ((nch - last * cpw) // 8) * 8
  rem = nch - last * cpw - nlast
  mesh = plsc.VectorSubcoreMesh(core_axis_name="c", subcore_axis_name="s")

  @functools.partial(
      pl.kernel,
      out_type=jax.ShapeDtypeStruct((e, d), jnp.float32),
      mesh=mesh,
      scratch_types=[
          pltpu.VMEM((cpw, cg), jnp.int32),
          pltpu.VMEM((cpw, cg), jnp.int32),
          pltpu.VMEM((cg,), jnp.int32),
          pltpu.VMEM((cg,), jnp.int32),
          pltpu.VMEM((cg, d), jnp.float32),
          pltpu.VMEM((cg, d), jnp.float32),
          pltpu.VMEM((cg, d), jnp.float32),
          pltpu.VMEM((cg, d), jnp.float32),
          pltpu.SemaphoreType.DMA,
          pltpu.SemaphoreType.DMA,
          pltpu.SemaphoreType.DMA,
          pltpu.SemaphoreType.DMA,
      ],
  )
  def k(table_h, src_h, dst_h, xsrc_h, xdst_h, o_h,
        srcb, dstb, xsrc, xdst, ra0, rb0, ra1, rb1, sa0, sb0, sa1, sb1):
    wid = lax.axis_index("s") * _NC + lax.axis_index("c")
    crow = wid * cpw
    ebase = crow * cg
    pairs = jnp.where(wid == last, nlast // 2, cpw // 2)

    @pl.when(wid < last)
    def _prefetch_full():
      pltpu.sync_copy(src_h.at[pl.ds(crow, cpw)], srcb)
      pltpu.sync_copy(dst_h.at[pl.ds(crow, cpw)], dstb)

    @pl.when(wid == last)
    def _prefetch_part():
      pltpu.sync_copy(src_h.at[pl.ds(crow, nlast)], srcb.at[pl.ds(0, nlast)])
      pltpu.sync_copy(dst_h.at[pl.ds(crow, nlast)], dstb.at[pl.ds(0, nlast)])

    def start(i, ra, rb, sa, sb):
      pltpu.async_copy(table_h.at[srcb.at[i]], ra, sa)
      pltpu.async_copy(table_h.at[dstb.at[i]], rb, sb)

    def psum(ra, rb):
      def row(r, carry):
        for kk in range(d // 16):
          sl = pl.ds(16 * kk, 16)
          ra[r, sl] = ra[r, sl] + rb[r, sl]
        return carry

      lax.fori_loop(0, cg, row, 0)

    def finish(i, ra, rb, sa, sb):
      off = ebase + i * cg
      pltpu.make_async_copy(table_h.at[srcb.at[i]], ra, sa).wait()
      pltpu.make_async_copy(table_h.at[dstb.at[i]], rb, sb).wait()
      psum(ra, rb)
      pltpu.sync_copy(ra, o_h.at[pl.ds(off, cg)])

    start(0, ra0, rb0, sa0, sb0)

    def pair(j, carry):
      i0 = 2 * j
      start(i0 + 1, ra1, rb1, sa1, sb1)
      finish(i0, ra0, rb0, sa0, sb0)

      @pl.when(j < pairs - 1)
      def _start_next():
        start(i0 + 2, ra0, rb0, sa0, sb0)

      finish(i0 + 1, ra1, rb1, sa1, sb1)
      return carry

    lax.fori_loop(0, pairs, pair, 0)

    @pl.when(wid < rem)
    def _extra():
      xoff = (nch - rem + wid) * cg
      pltpu.sync_copy(xsrc_h.at[pl.ds(wid * cg, cg)], xsrc)
      pltpu.sync_copy(xdst_h.at[pl.ds(wid * cg, cg)], xdst)
      cpa = pltpu.async_copy(table_h.at[xsrc], ra0, sa0)
      cpb = pltpu.async_copy(table_h.at[xdst], rb0, sb0)
      cpa.wait()
      cpb.wait()
      psum(ra0, rb0)
      pltpu.sync_copy(ra0, o_h.at[pl.ds(xoff, cg)])

  return k(table, src2, dst2, xsrc1, xdst1)


def _sc_segsum(feat, dst2, xdst1, n):
  e, d = feat.shape
  nch, cg = dst2.shape
  cpw = -(-nch // _NW)
  cpw += (-cpw) % 8
  last = _NW - 1
  nlast = ((nch - last * cpw) // 8) * 8
  rem = nch - last * cpw - nlast
  rps = (n // (_NS * 8)) * 8
  ztail = n - rps * _NS
  zrows = 16
  mesh = plsc.VectorSubcoreMesh(core_axis_name="c", subcore_axis_name="s")

  @functools.partial(
      pl.kernel,
      out_type=jax.ShapeDtypeStruct((2 * n, d), jnp.float32),
      mesh=mesh,
      scratch_types=[
          pltpu.VMEM((cpw, cg), jnp.int32),
          pltpu.VMEM((cg,), jnp.int32),
          pltpu.VMEM((cg, d), jnp.float32),
          pltpu.VMEM((cg, d), jnp.float32),
          pltpu.VMEM((zrows, d), jnp.float32),
          pltpu.VMEM_SHARED((n, d), jnp.float32),
          pltpu.SemaphoreType.DMA,
          pltpu.SemaphoreType.DMA,
      ],
  )
  def k(feat_h, dst_h, xdst_h, sums_h, dstb, xdst, rows0, rows1, zbuf_v,
        acc_s, sm0, sm1):
    cid = lax.axis_index("c")
    sid = lax.axis_index("s")
    wid = sid * _NC + cid
    crow = wid * cpw
    ebase = crow * cg
    pairs = jnp.where(wid == last, nlast // 2, cpw // 2)

    zero16 = jnp.zeros((16,), jnp.float32)

    def fill_z(r, carry):
      for kk in range(d // 16):
        zbuf_v[r, pl.ds(16 * kk, 16)] = zero16
      return carry

    lax.fori_loop(0, zrows, fill_z, 0)

    rbase = sid * rps
    for j in range(rps // zrows):
      pltpu.sync_copy(zbuf_v, acc_s.at[pl.ds(rbase + j * zrows, zrows)])

    @pl.when(sid == _NS - 1)
    def _zero_tail():
      pltpu.sync_copy(zbuf_v.at[pl.ds(0, ztail)],
                      acc_s.at[pl.ds(rps * _NS, ztail)])

    @pl.when(wid < last)
    def _prefetch_full():
      pltpu.sync_copy(dst_h.at[pl.ds(crow, cpw)], dstb)

    @pl.when(wid == last)
    def _prefetch_part():
      pltpu.sync_copy(dst_h.at[pl.ds(crow, nlast)], dstb.at[pl.ds(0, nlast)])

    plsc.subcore_barrier()

    def start(i, rows, sm):
      pltpu.async_copy(feat_h.at[pl.ds(ebase + i * cg, cg)], rows, sm)

    def finish(i, rows, sm):
      pltpu.make_async_copy(feat_h.at[pl.ds(ebase + i * cg, cg)],
                            rows, sm).wait()
      pltpu.sync_copy(rows, acc_s.at[dstb.at[i]], add=True)

    start(0, rows0, sm0)

    def pair(j, carry):
      i0 = 2 * j
      start(i0 + 1, rows1, sm1)
      finish(i0, rows0, sm0)

      @pl.when(j < pairs - 1)
      def _start_next():
        start(i0 + 2, rows0, sm0)

      finish(i0 + 1, rows1, sm1)
      return carry

    lax.fori_loop(0, pairs, pair, 0)

    @pl.when(wid < rem)
    def _extra():
      xoff = (nch - rem + wid) * cg
      pltpu.sync_copy(xdst_h.at[pl.ds(wid * cg, cg)], xdst)
      pltpu.sync_copy(feat_h.at[pl.ds(xoff, cg)], rows0)
      pltpu.sync_copy(rows0, acc_s.at[xdst], add=True)

    plsc.subcore_barrier()

    obase = cid * n + rbase
    pltpu.sync_copy(acc_s.at[pl.ds(rbase, rps)], sums_h.at[pl.ds(obase, rps)])

    @pl.when(sid == _NS - 1)
    def _write_tail():
      tbase = rps * _NS
      pltpu.sync_copy(acc_s.at[pl.ds(tbase, ztail)],
                      sums_h.at[pl.ds(cid * n + tbase, ztail)])

  return k(feat, dst2, xdst1)


def _sc_counts(dst2, xdst1, n):
  nch, cg = dst2.shape
  w = 128
  cpw = -(-nch // _NW)
  cpw += (-cpw) % 8
  last = _NW - 1
  nlast = ((nch - last * cpw) // 8) * 8
  rem = nch - last * cpw - nlast
  rps = (n // (_NS * 8)) * 8
  ztail = n - rps * _NS
  zrows = 48
  mesh = plsc.VectorSubcoreMesh(core_axis_name="c", subcore_axis_name="s")

  @functools.partial(
      pl.kernel,
      out_type=jax.ShapeDtypeStruct((2 * n, w), jnp.float32),
      mesh=mesh,
      scratch_types=[
          pltpu.VMEM((cpw, cg), jnp.int32),
          pltpu.VMEM((cg,), jnp.int32),
          pltpu.VMEM((cg, w), jnp.float32),
          pltpu.VMEM((zrows, w), jnp.float32),
          pltpu.VMEM_SHARED((n, w), jnp.float32),
      ],
  )
  def k(dst_h, xdst_h, cnts_h, dstb, xdst, ones_v, zbuf_v, cnt_s):
    cid = lax.axis_index("c")
    sid = lax.axis_index("s")
    wid = sid * _NC + cid
    crow = wid * cpw

    zero16 = jnp.zeros((16,), jnp.float32)
    one16 = jnp.ones((16,), jnp.float32)

    def fill_z(r, carry):
      for kk in range(w // 16):
        zbuf_v[r, pl.ds(16 * kk, 16)] = zero16
      return carry

    lax.fori_loop(0, zrows, fill_z, 0)

    def fill_o(r, carry):
      for kk in range(w // 16):
        ones_v[r, pl.ds(16 * kk, 16)] = one16
      return carry

    lax.fori_loop(0, cg, fill_o, 0)

    rbase = sid * rps
    for j in range(rps // zrows):
      pltpu.sync_copy(zbuf_v, cnt_s.at[pl.ds(rbase + j * zrows, zrows)])

    @pl.when(sid == _NS - 1)
    def _zero_tail():
      pltpu.sync_copy(zbuf_v.at[pl.ds(0, ztail)],
                      cnt_s.at[pl.ds(rps * _NS, ztail)])

    nloc = jnp.where(wid == last, nlast, cpw)

    @pl.when(wid < last)
    def _prefetch_full():
      pltpu.sync_copy(dst_h.at[pl.ds(crow, cpw)], dstb)

    @pl.when(wid == last)
    def _prefetch_part():
      pltpu.sync_copy(dst_h.at[pl.ds(crow, nlast)], dstb.at[pl.ds(0, nlast)])

    plsc.subcore_barrier()

    def step(i, carry):
      pltpu.sync_copy(ones_v, cnt_s.at[dstb.at[i]], add=True)
      return carry

    lax.fori_loop(0, nloc, step, 0)

    @pl.when(wid < rem)
    def _extra():
      pltpu.sync_copy(xdst_h.at[pl.ds(wid * cg, cg)], xdst)
      pltpu.sync_copy(ones_v, cnt_s.at[xdst], add=True)

    plsc.subcore_barrier()

    obase = cid * n + rbase
    pltpu.sync_copy(cnt_s.at[pl.ds(rbase, rps)], cnts_h.at[pl.ds(obase, rps)])

    @pl.when(sid == _NS - 1)
    def _write_tail():
      tbase = rps * _NS
      pltpu.sync_copy(cnt_s.at[pl.ds(tbase, ztail)],
                      cnts_h.at[pl.ds(cid * n + tbase, ztail)])

  return k(dst2, xdst1)


def _dot(a, b):
  return jnp.dot(a, b, preferred_element_type=jnp.float32)


def _tc_embed(h, we, be, w1n, b1n, a_s):
  n, f = h.shape
  d = we.shape[1]
  grid = n // _BN

  def body(a_ref, h_ref, we_ref, be_ref, w1_ref, b1_ref,
           h0_ref, fn1_ref, fn1s_ref):
    h0 = _dot(h_ref[...], we_ref[...]) + be_ref[...]
    h0_ref[...] = h0
    fn1 = jnp.maximum(_dot(h0, w1_ref[...]) + b1_ref[...], 0.0)
    fn1_ref[...] = fn1
    fn1s_ref[...] = fn1 * a_ref[0, 0]

  return pl.pallas_call(
      body,
      grid=(grid,),
      in_specs=[
          pl.BlockSpec(memory_space=pltpu.SMEM),
          pl.BlockSpec((_BN, f), lambda i: (i, 0)),
          pl.BlockSpec((f, d), lambda i: (0, 0)),
          pl.BlockSpec((1, d), lambda i: (0, 0)),
          pl.BlockSpec((d, d), lambda i: (0, 0)),
          pl.BlockSpec((1, d), lambda i: (0, 0)),
      ],
      out_specs=[pl.BlockSpec((_BN, d), lambda i: (i, 0))] * 3,
      out_shape=[jax.ShapeDtypeStruct((n, d), jnp.float32)] * 3,
  )(a_s, h, we, be.reshape(1, d), w1n, b1n.reshape(1, d))


def _tc_mm1(g, w, b):
  e, d = g.shape
  grid = e // _BE

  def body(g_ref, w_ref, b_ref, o_ref):
    o_ref[...] = jnp.maximum(_dot(g_ref[...], w_ref[...]) + b_ref[...], 0.0)

  return pl.pallas_call(
      body,
      grid=(grid,),
      in_specs=[
          pl.BlockSpec((_BE, d), lambda i: (i, 0)),
          pl.BlockSpec((d, d), lambda i: (0, 0)),
          pl.BlockSpec((1, d), lambda i: (0, 0)),
      ],
      out_specs=pl.BlockSpec((_BE, d), lambda i: (i, 0)),
      out_shape=jax.ShapeDtypeStruct((e, d), jnp.float32),
  )(g, w, b.reshape(1, d))


def _tc_mm2_mlp56(s2, g2, fe1, g3, w3e, b3e,
                  w1, b1, w2, b2, w3, b3, w4, b4, wo, bo):
  e, d = g2.shape
  nc = wo.shape[1]
  grid = e // _BE

  def body(s_ref, g2_ref, fe1_ref, g3_ref,
           w3e_ref, b3e_ref,
           w1_ref, b1_ref, w2_ref, b2_ref, w3_ref, b3_ref, w4_ref, b4_ref,
           wo_ref, bo_ref, fe3_ref, o_ref):
    x = g2_ref[...] + s_ref[0, 0] * fe1_ref[...]
    fe3 = jnp.maximum(_dot(x, w3e_ref[...]) + b3e_ref[...], 0.0)
    fe3_ref[...] = fe3
    y = g3_ref[...] + s_ref[0, 1] * fe3
    o_ref[...] = _mlp56(y, w1_ref[...], b1_ref[...], w2_ref[...], b2_ref[...],
                        w3_ref[...], b3_ref[...], w4_ref[...], b4_ref[...],
                        wo_ref[...], bo_ref[...])

  espec = pl.BlockSpec((_BE, d), lambda i: (i, 0))
  wspec = pl.BlockSpec((d, d), lambda i: (0, 0))
  bspec = pl.BlockSpec((1, d), lambda i: (0, 0))
  return pl.pallas_call(
      body,
      grid=(grid,),
      in_specs=[
          pl.BlockSpec(memory_space=pltpu.SMEM),
          espec, espec, espec,
          wspec, bspec,
          wspec, bspec, wspec, bspec, wspec, bspec, wspec, bspec,
          pl.BlockSpec((d, nc), lambda i: (0, 0)),
          pl.BlockSpec((1, nc), lambda i: (0, 0)),
      ],
      out_specs=[pl.BlockSpec((_BE, d), lambda i: (i, 0)),
                 pl.BlockSpec((_BE, nc), lambda i: (i, 0))],
      out_shape=[jax.ShapeDtypeStruct((e, d), jnp.float32),
                 jax.ShapeDtypeStruct((e, nc), jnp.float32)],
  )(s2, g2, fe1, g3, w3e, b3e.reshape(1, d),
    w1, b1.reshape(1, d), w2, b2.reshape(1, d), w3, b3.reshape(1, d),
    w4, b4.reshape(1, d), wo, bo.reshape(1, nc))


def _mlp56(x, w1, b1, w2, b2, w3, b3, w4, b4, wo, bo):
  x = jnp.maximum(_dot(x, w1) + b1, 0.0)
  x = jnp.maximum(_dot(x, w2) + b2, 0.0)
  x = jnp.maximum(_dot(x, w3) + b3, 0.0)
  x = jnp.maximum(_dot(x, w4) + b4, 0.0)
  return _dot(x, wo) + bo


def _tc_mlp56_edges(s, ga, gb, fe, w1, b1, w2, b2, w3, b3, w4, b4, wo, bo):
  e, d = ga.shape
  nc = wo.shape[1]
  grid = e // _BE

  def body(s_ref, ga_ref, gb_ref, fe_ref,
           w1_ref, b1_ref, w2_ref, b2_ref, w3_ref, b3_ref, w4_ref, b4_ref,
           wo_ref, bo_ref, o_ref):
    x = ga_ref[...] + gb_ref[...] + s_ref[0, 0] * fe_ref[...]
    o_ref[...] = _mlp56(x, w1_ref[...], b1_ref[...], w2_ref[...], b2_ref[...],
                        w3_ref[...], b3_ref[...], w4_ref[...], b4_ref[...],
                        wo_ref[...], bo_ref[...])

  wspec = pl.BlockSpec((d, d), lambda i: (0, 0))
  bspec = pl.BlockSpec((1, d), lambda i: (0, 0))
  espec = pl.BlockSpec((_BE, d), lambda i: (i, 0))
  return pl.pallas_call(
      body,
      grid=(grid,),
      in_specs=[
          pl.BlockSpec(memory_space=pltpu.SMEM),
          espec, espec, espec,
          wspec, bspec, wspec, bspec, wspec, bspec, wspec, bspec,
          pl.BlockSpec((d, nc), lambda i: (0, 0)),
          pl.BlockSpec((1, nc), lambda i: (0, 0)),
      ],
      out_specs=pl.BlockSpec((_BE, nc), lambda i: (i, 0)),
      out_shape=jax.ShapeDtypeStruct((e, nc), jnp.float32),
  )(s, ga, gb, fe, w1, b1.reshape(1, d), w2, b2.reshape(1, d),
    w3, b3.reshape(1, d), w4, b4.reshape(1, d), wo, bo.reshape(1, nc))


def _tc_node_mid(scal, fn1, sums, cnts, w3n, b3n):
  n, d = fn1.shape
  grid = n // _BN
  nblocks = n // _BN

  def body(s_ref, fn1_ref, s0_ref, s1_ref, c0_ref, c1_ref, w_ref, b_ref,
           fn3_ref, fn3s_ref):
    cnt = c0_ref[:, 0:1] + c1_ref[:, 0:1]
    segm = (s0_ref[...] + s1_ref[...]) / jnp.maximum(cnt, 1.0)
    new_n = s_ref[0, 0] * fn1_ref[...] + s_ref[0, 1] * segm
    fn3 = jnp.maximum(_dot(new_n, w_ref[...]) + b_ref[...], 0.0)
    fn3_ref[...] = fn3
    fn3s_ref[...] = fn3 * s_ref[0, 2]

  return pl.pallas_call(
      body,
      grid=(grid,),
      in_specs=[
          pl.BlockSpec(memory_space=pltpu.SMEM),
          pl.BlockSpec((_BN, d), lambda i: (i, 0)),
          pl.BlockSpec((_BN, d), lambda i: (i, 0)),
          pl.BlockSpec((_BN, d), lambda i, nb=nblocks: (nb + i, 0)),
          pl.BlockSpec((_BN, 128), lambda i: (i, 0)),
          pl.BlockSpec((_BN, 128), lambda i, nb=nblocks: (nb + i, 0)),
          pl.BlockSpec((d, d), lambda i: (0, 0)),
          pl.BlockSpec((1, d), lambda i: (0, 0)),
      ],
      out_specs=[pl.BlockSpec((_BN, d), lambda i: (i, 0))] * 2,
      out_shape=[jax.ShapeDtypeStruct((n, d), jnp.float32)] * 2,
  )(scal, fn1, sums, sums, cnts, cnts, w3n, b3n.reshape(1, d))


def _tc_node_final(scal, fn3, sums, cnts,
                   w1, b1, w2, b2, w3, b3, w4, b4, wo, bo):
  n, d = fn3.shape
  nc = wo.shape[1]
  grid = n // _BN
  nblocks = n // _BN

  def body(s_ref, fn3_ref, s0_ref, s1_ref, c0_ref, c1_ref,
           w1_ref, b1_ref, w2_ref, b2_ref, w3_ref, b3_ref, w4_ref, b4_ref,
           wo_ref, bo_ref, o_ref):
    cnt = c0_ref[:, 0:1] + c1_ref[:, 0:1]
    segm = (s0_ref[...] + s1_ref[...]) / jnp.maximum(cnt, 1.0)
    new_n = s_ref[0, 0] * fn3_ref[...] + s_ref[0, 1] * segm
    o_ref[...] = _mlp56(new_n, w1_ref[...], b1_ref[...], w2_ref[...],
                        b2_ref[...], w3_ref[...], b3_ref[...], w4_ref[...],
                        b4_ref[...], wo_ref[...], bo_ref[...])

  wspec = pl.BlockSpec((d, d), lambda i: (0, 0))
  bspec = pl.BlockSpec((1, d), lambda i: (0, 0))
  return pl.pallas_call(
      body,
      grid=(grid,),
      in_specs=[
          pl.BlockSpec(memory_space=pltpu.SMEM),
          pl.BlockSpec((_BN, d), lambda i: (i, 0)),
          pl.BlockSpec((_BN, d), lambda i: (i, 0)),
          pl.BlockSpec((_BN, d), lambda i, nb=nblocks: (nb + i, 0)),
          pl.BlockSpec((_BN, 128), lambda i: (i, 0)),
          pl.BlockSpec((_BN, 128), lambda i, nb=nblocks: (nb + i, 0)),
          wspec, bspec, wspec, bspec, wspec, bspec, wspec, bspec,
          pl.BlockSpec((d, nc), lambda i: (0, 0)),
          pl.BlockSpec((1, nc), lambda i: (0, 0)),
      ],
      out_specs=pl.BlockSpec((_BN, nc), lambda i: (i, 0)),
      out_shape=jax.ShapeDtypeStruct((n, nc), jnp.float32),
  )(scal, fn3, sums, sums, cnts, cnts, w1, b1.reshape(1, d), w2,
    b2.reshape(1, d), w3, b3.reshape(1, d), w4, b4.reshape(1, d),
    wo, bo.reshape(1, nc))


def kernel(h, edge_index, W_embed, b_embed, W1n, b1n, W1e, b1e,
           w2_nn, w2_ne, w2_en, w2_ee, W3n, b3n, W3e, b3e,
           w4_nn, w4_ne, w4_en, w4_ee, W5_1, b5_1, W5_2, b5_2,
           W5_3, b5_3, W5_4, b5_4, W5_out, b5_out):
  n = h.shape[0]
  src2 = edge_index[0].reshape(-1, 128)
  dst2 = edge_index[1].reshape(-1, 128)
  nrem = 512
  xs1 = edge_index[0, -nrem:]
  xd1 = edge_index[1, -nrem:]

  a2 = (0.5 * w2_en).reshape(1, 1)
  h0, fn1, fn1s = _tc_embed(h, W_embed, b_embed, W1n, b1n, a2)

  cnts = _sc_counts(dst2, xd1, n)
  g1 = _sc_gather_pair(h0, src2, dst2, xs1, xd1)
  fe1 = _tc_mm1(g1, 0.5 * W1e, b1e)
  sums1 = _sc_segsum(fe1, dst2, xd1, n)

  scal_mid = jnp.concatenate([w2_nn, w2_ne, 0.5 * w4_en]).reshape(1, 3)
  fn3, fn3s = _tc_node_mid(scal_mid, fn1, sums1, cnts, W3n, b3n)

  g2 = _sc_gather_pair(fn1s, src2, dst2, xs1, xd1)
  g3 = _sc_gather_pair(fn3s, src2, dst2, xs1, xd1)
  s2 = jnp.concatenate([w2_ee, w4_ee]).reshape(1, 2)
  fe3, out_e = _tc_mm2_mlp56(s2, g2, fe1, g3, W3e, b3e,
                             W5_1, b5_1, W5_2, b5_2, W5_3, b5_3, W5_4, b5_4,
                             W5_out, b5_out)
  sums2 = _sc_segsum(fe3, dst2, xd1, n)

  scal_fin = jnp.concatenate([w4_nn, w4_ne]).reshape(1, 2)
  out_n = _tc_node_final(scal_fin, fn3, sums2, cnts,
                         W5_1, b5_1, W5_2, b5_2, W5_3, b5_3, W5_4, b5_4,
                         W5_out, b5_out)
  return out_n, out_e
